# Initial kernel scaffold; baseline (speedup 1.0000x reference)
import jax
import jax.numpy as jnp
from jax.experimental import pallas as pl


def kernel(x, edge_index, batch, emb0, emb1, emb2, emb3, emb4, emb5, emb6, emb7, emb8, W0, b0, W1, b1, W2, b2, bn0_g, bn0_b, bn1_g, bn1_b, lin_W, lin_b):
    raise NotImplementedError("write your pallas kernel here")


# speedup vs baseline: 23.1297x; 23.1297x over previous
import functools

import numpy as np
import jax
import jax.numpy as jnp
from jax import lax
from jax.experimental import pallas as pl
from jax.experimental.pallas import tpu as pltpu
from jax.experimental.pallas import tpu_sc as plsc

N = 10000
NP = 10240
E = 320000
H = 128
G = 128
K = 80
NT = 32
CPT = E // NT // K
NGRP = 5
GCPT = CPT // NGRP
CPT_DEG = E // 16 // K
ROWS_T = NP // 16
BLK = 1024
ATOM_DIMS = [119, 4, 12, 12, 10, 6, 6, 2, 2]
TOT = sum(ATOM_DIMS)
TPAD = 176
BN_SCALE = float(1.0 / np.sqrt(1.0 + 1e-5))

def _deg_body(dst3d, deg_out, idx_v, ones_v, zero_v, acc_sh):
    c = lax.axis_index("c")
    s = lax.axis_index("s")

    @pl.when(c == 0)
    def _():
        for i in range(K // 16):
            ones_v[pl.ds(i * 16, 16)] = jnp.ones((16,), jnp.float32)
        for i in range(ROWS_T // 16):
            zero_v[pl.ds(i * 16, 16)] = jnp.zeros((16,), jnp.float32)
        pltpu.sync_copy(zero_v, acc_sh.at[pl.ds(s * ROWS_T, ROWS_T)])
        plsc.subcore_barrier()
        pltpu.sync_copy(dst3d.at[s], idx_v)

        def body(j, carry):
            pltpu.sync_copy(ones_v, acc_sh.at[idx_v.at[j]], add=True)
            return carry

        lax.fori_loop(0, CPT_DEG, body, 0)
        plsc.subcore_barrier()
        pltpu.sync_copy(acc_sh.at[pl.ds(s * ROWS_T, ROWS_T)],
                        deg_out.at[pl.ds(s * ROWS_T, ROWS_T)])


def _agg_body(p_hbm, src4d, dst4d, out, sidx, didx, buf0, buf1,
              acc_sh, sem0, sem1):
    c = lax.axis_index("c")
    s = lax.axis_index("s")
    wid = c * 16 + s

    def zb(i, carry):
        buf0[i // 8, pl.ds((i % 8) * 16, 16)] = jnp.zeros((16,), jnp.float32)
        return carry

    lax.fori_loop(0, K * 8, zb, 0)
    for kk in range(ROWS_T // K):
        pltpu.sync_copy(buf0, acc_sh.at[pl.ds(s * ROWS_T + kk * K, K)])
    plsc.subcore_barrier()

    bufs = (buf0, buf1)
    sems = (sem0, sem1)
    for g in range(NGRP):
        pltpu.sync_copy(src4d.at[wid, g], sidx)
        pltpu.sync_copy(dst4d.at[wid, g], didx)
        pltpu.async_copy(p_hbm.at[sidx.at[0]], buf0, sem0)

        def outer(jj, carry):
            for b in range(2):
                j = jj * 2 + b

                @pl.when(j < GCPT)
                def _():
                    @pl.when(j + 1 < GCPT)
                    def _():
                        pltpu.async_copy(p_hbm.at[sidx.at[j + 1]],
                                         bufs[(b + 1) % 2], sems[(b + 1) % 2])

                    pltpu.make_async_copy(p_hbm.at[sidx.at[j]], bufs[b],
                                          sems[b]).wait()
                    pltpu.sync_copy(bufs[b], acc_sh.at[didx.at[j]], add=True)
            return carry

        lax.fori_loop(0, (GCPT + 1) // 2, outer, 0)
    plsc.subcore_barrier()
    pltpu.sync_copy(acc_sh.at[pl.ds(s * ROWS_T, ROWS_T)],
                    out.at[c, pl.ds(s * ROWS_T, ROWS_T)])


@functools.cache
def _sc_kernels():
    mesh = plsc.VectorSubcoreMesh(core_axis_name="c", subcore_axis_name="s",
                                  num_cores=2, num_subcores=16)
    deg_kernel = pl.kernel(
        _deg_body,
        out_type=jax.ShapeDtypeStruct((NP,), jnp.float32),
        mesh=mesh,
        scratch_types=[
            pltpu.VMEM((CPT_DEG, K), jnp.int32),
            pltpu.VMEM((K,), jnp.float32),
            pltpu.VMEM((ROWS_T,), jnp.float32),
            pltpu.VMEM_SHARED((NP,), jnp.float32),
        ],
    )
    agg_kernel = pl.kernel(
        _agg_body,
        out_type=jax.ShapeDtypeStruct((2, NP, H), jnp.float32),
        mesh=mesh,
        scratch_types=[
            pltpu.VMEM((GCPT, K), jnp.int32),
            pltpu.VMEM((GCPT, K), jnp.int32),
            pltpu.VMEM((K, H), jnp.float32),
            pltpu.VMEM((K, H), jnp.float32),
            pltpu.VMEM_SHARED((NP, H), jnp.float32),
            pltpu.SemaphoreType.DMA,
            pltpu.SemaphoreType.DMA,
        ],
    )
    return deg_kernel, agg_kernel


def _emb_body(x_ref, deg_ref, tcat_ref, w0_ref, p0_ref, dinv_ref):
    m0 = lax.dot_general(tcat_ref[...], w0_ref[...], (((1,), (0,)), ((), ())),
                         preferred_element_type=jnp.float32)
    xt = x_ref[...]
    iota = lax.broadcasted_iota(jnp.int32, (BLK, TPAD), 1)
    cmat = jnp.zeros((BLK, TPAD), jnp.float32)
    off = 0
    for i, d in enumerate(ATOM_DIMS):
        cmat = cmat + jnp.where(iota == xt[:, i:i + 1] + off, 1.0, 0.0)
        off += d
    h = lax.dot_general(cmat, m0, (((1,), (0,)), ((), ())),
                        preferred_element_type=jnp.float32)
    dinv = lax.rsqrt(deg_ref[...] + 1.0)
    p0_ref[...] = dinv * h
    dinv_ref[...] = dinv


_emb_call = pl.pallas_call(
    _emb_body,
    grid=(NP // BLK,),
    in_specs=[
        pl.BlockSpec((BLK, 16), lambda i: (i, 0)),
        pl.BlockSpec((BLK, 1), lambda i: (i, 0)),
        pl.BlockSpec((TPAD, H), lambda i: (0, 0)),
        pl.BlockSpec((H, H), lambda i: (0, 0)),
    ],
    out_specs=[
        pl.BlockSpec((BLK, H), lambda i: (i, 0)),
        pl.BlockSpec((BLK, 1), lambda i: (i, 0)),
    ],
    out_shape=[
        jax.ShapeDtypeStruct((NP, H), jnp.float32),
        jax.ShapeDtypeStruct((NP, 1), jnp.float32),
    ],
)


def _layer_body(acc_ref, p_ref, dinv_ref, b_ref, bng_ref, bnb_ref, w_ref,
                out_ref):
    agg = acc_ref[0] + acc_ref[1] + p_ref[...]
    conv = dinv_ref[...] * agg + b_ref[...]
    h = jnp.maximum(conv * (BN_SCALE * bng_ref[...]) + bnb_ref[...], 0.0)
    out_ref[...] = dinv_ref[...] * lax.dot_general(
        h, w_ref[...], (((1,), (0,)), ((), ())),
        preferred_element_type=jnp.float32)


_layer_call = pl.pallas_call(
    _layer_body,
    grid=(NP // BLK,),
    in_specs=[
        pl.BlockSpec((2, BLK, H), lambda i: (0, i, 0)),
        pl.BlockSpec((BLK, H), lambda i: (i, 0)),
        pl.BlockSpec((BLK, 1), lambda i: (i, 0)),
        pl.BlockSpec((1, H), lambda i: (0, 0)),
        pl.BlockSpec((1, H), lambda i: (0, 0)),
        pl.BlockSpec((1, H), lambda i: (0, 0)),
        pl.BlockSpec((H, H), lambda i: (0, 0)),
    ],
    out_specs=pl.BlockSpec((BLK, H), lambda i: (i, 0)),
    out_shape=jax.ShapeDtypeStruct((NP, H), jnp.float32),
)


def _final_body(acc_ref, p_ref, dinv_ref, b2_ref, batch_ref, linw_ref,
                linb_ref, out_ref, sums_ref, cnts_ref):
    i = pl.program_id(0)

    @pl.when(i == 0)
    def _():
        sums_ref[...] = jnp.zeros_like(sums_ref)
        cnts_ref[...] = jnp.zeros_like(cnts_ref)

    h = dinv_ref[...] * (acc_ref[0] + acc_ref[1] + p_ref[...]) + b2_ref[...]
    ind = jnp.where(
        batch_ref[...] == lax.broadcasted_iota(jnp.int32, (BLK, G), 1),
        1.0, 0.0)
    sums_ref[...] += lax.dot_general(ind, h, (((0,), (0,)), ((), ())),
                                     preferred_element_type=jnp.float32)
    cnts_ref[...] += lax.dot_general(ind, jnp.ones((BLK, 1), jnp.float32),
                                     (((0,), (0,)), ((), ())),
                                     preferred_element_type=jnp.float32)

    @pl.when(i == pl.num_programs(0) - 1)
    def _():
        feats = sums_ref[...] / jnp.maximum(cnts_ref[...], 1.0)
        z = lax.dot_general(feats, linw_ref[...], (((1,), (0,)), ((), ())),
                            preferred_element_type=jnp.float32) + linb_ref[...]
        out_ref[...] = 1.0 / (1.0 + jnp.exp(-z))


_final_call = pl.pallas_call(
    _final_body,
    grid=(NP // BLK,),
    in_specs=[
        pl.BlockSpec((2, BLK, H), lambda i: (0, i, 0)),
        pl.BlockSpec((BLK, H), lambda i: (i, 0)),
        pl.BlockSpec((BLK, 1), lambda i: (i, 0)),
        pl.BlockSpec((1, H), lambda i: (0, 0)),
        pl.BlockSpec((BLK, 1), lambda i: (i, 0)),
        pl.BlockSpec((H, 1), lambda i: (0, 0)),
        pl.BlockSpec((1, 1), lambda i: (0, 0)),
    ],
    out_specs=pl.BlockSpec((G, 1), lambda i: (0, 0)),
    out_shape=jax.ShapeDtypeStruct((G, 1), jnp.float32),
    scratch_shapes=[
        pltpu.VMEM((G, H), jnp.float32),
        pltpu.VMEM((G, 1), jnp.float32),
    ],
)


def kernel(x, edge_index, batch, emb0, emb1, emb2, emb3, emb4, emb5, emb6,
           emb7, emb8, W0, b0, W1, b1, W2, b2, bn0_g, bn0_b, bn1_g, bn1_b,
           lin_W, lin_b):
    xp = jnp.pad(x, ((0, NP - N), (0, 16 - 9)))
    src4d = edge_index[0].reshape(NT, NGRP, GCPT, K)
    dst4d = edge_index[1].reshape(NT, NGRP, GCPT, K)
    dst3d_deg = edge_index[1].reshape(16, CPT_DEG, K)
    batch_p = jnp.pad(batch, (0, NP - N), constant_values=G).reshape(NP, 1)
    tcat = jnp.pad(
        jnp.concatenate([emb0, emb1, emb2, emb3, emb4, emb5, emb6, emb7,
                         emb8], axis=0), ((0, TPAD - TOT), (0, 0)))

    deg_kernel, agg_kernel = _sc_kernels()
    deg = deg_kernel(dst3d_deg).reshape(NP, 1)
    p0, dinv = _emb_call(xp, deg, tcat, W0)
    acc0 = agg_kernel(p0, src4d, dst4d)
    p1 = _layer_call(acc0, p0, dinv, b0.reshape(1, H), bn0_g.reshape(1, H),
                     bn0_b.reshape(1, H), W1)
    acc1 = agg_kernel(p1, src4d, dst4d)
    p2 = _layer_call(acc1, p1, dinv, b1.reshape(1, H), bn1_g.reshape(1, H),
                     bn1_b.reshape(1, H), W2)
    acc2 = agg_kernel(p2, src4d, dst4d)
    out = _final_call(acc2, p2, dinv, b2.reshape(1, H), batch_p, lin_W,
                      lin_b.reshape(1, 1))
    return out


# speedup vs baseline: 25.8952x; 1.1196x over previous
import functools

import numpy as np
import jax
import jax.numpy as jnp
from jax import lax
from jax.experimental import pallas as pl
from jax.experimental.pallas import tpu as pltpu
from jax.experimental.pallas import tpu_sc as plsc

N = 10000
NP = 10240
E = 320000
H = 128
G = 128
K = 80
NT = 32
CPT = E // NT // K
NGRP = 5
GCPT = CPT // NGRP
CPT_DEG = E // 16 // K
ROWS_T = NP // 16
BLK = 1024
ATOM_DIMS = [119, 4, 12, 12, 10, 6, 6, 2, 2]
TOT = sum(ATOM_DIMS)
TPAD = 176
BN_SCALE = float(1.0 / np.sqrt(1.0 + 1e-5))

def _deg_body(dst3d, deg_out, idx_v, ones_v, zero_v, acc_sh):
    c = lax.axis_index("c")
    s = lax.axis_index("s")

    @pl.when(c == 0)
    def _():
        for i in range(K // 16):
            ones_v[pl.ds(i * 16, 16)] = jnp.ones((16,), jnp.float32)
        for i in range(ROWS_T // 16):
            zero_v[pl.ds(i * 16, 16)] = jnp.zeros((16,), jnp.float32)
        pltpu.sync_copy(zero_v, acc_sh.at[pl.ds(s * ROWS_T, ROWS_T)])
        plsc.subcore_barrier()
        pltpu.sync_copy(dst3d.at[s], idx_v)

        def body(j, carry):
            pltpu.sync_copy(ones_v, acc_sh.at[idx_v.at[j]], add=True)
            return carry

        lax.fori_loop(0, CPT_DEG, body, 0)
        plsc.subcore_barrier()
        pltpu.sync_copy(acc_sh.at[pl.ds(s * ROWS_T, ROWS_T)],
                        deg_out.at[pl.ds(s * ROWS_T, ROWS_T)])


def _agg_body(p_hbm, src4d, dst4d, out, sidx, didx, buf0, buf1, buf2,
              acc_sh, gsem0, gsem1, gsem2, ssem0, ssem1, ssem2):
    c = lax.axis_index("c")
    s = lax.axis_index("s")
    wid = c * 16 + s

    def zb(i, carry):
        buf0[i // 8, pl.ds((i % 8) * 16, 16)] = jnp.zeros((16,), jnp.float32)
        return carry

    lax.fori_loop(0, K * 8, zb, 0)
    for kk in range(ROWS_T // K):
        pltpu.sync_copy(buf0, acc_sh.at[pl.ds(s * ROWS_T + kk * K, K)])
    plsc.subcore_barrier()

    bufs = (buf0, buf1, buf2)
    gsems = (gsem0, gsem1, gsem2)
    ssems = (ssem0, ssem1, ssem2)
    for g in range(NGRP):
        pltpu.sync_copy(src4d.at[wid, g], sidx)
        pltpu.sync_copy(dst4d.at[wid, g], didx)
        pltpu.async_copy(p_hbm.at[sidx.at[0]], buf0, gsem0)
        pltpu.async_copy(p_hbm.at[sidx.at[1]], buf1, gsem1)

        def outer(jj, carry):
            for b in range(3):
                j = jj * 3 + b

                @pl.when(j < GCPT)
                def _():
                    pltpu.make_async_copy(p_hbm.at[sidx.at[j]], bufs[b],
                                          gsems[b]).wait()
                    pltpu.async_copy(bufs[b], acc_sh.at[didx.at[j]],
                                     ssems[b], add=True)
                    bn = (b + 2) % 3

                    @pl.when(j + 2 < GCPT)
                    def _():
                        @pl.when(j >= 1)
                        def _():
                            pltpu.make_async_copy(
                                bufs[bn], acc_sh.at[didx.at[j]],
                                ssems[bn]).wait()

                        pltpu.async_copy(p_hbm.at[sidx.at[j + 2]],
                                         bufs[bn], gsems[bn])
            return carry

        lax.fori_loop(0, (GCPT + 2) // 3, outer, 0)
        for b in range(3):
            pltpu.make_async_copy(bufs[b], acc_sh.at[didx.at[0]],
                                  ssems[b]).wait()
    plsc.subcore_barrier()
    pltpu.sync_copy(acc_sh.at[pl.ds(s * ROWS_T, ROWS_T)],
                    out.at[c, pl.ds(s * ROWS_T, ROWS_T)])


@functools.cache
def _sc_kernels():
    mesh = plsc.VectorSubcoreMesh(core_axis_name="c", subcore_axis_name="s",
                                  num_cores=2, num_subcores=16)
    deg_kernel = pl.kernel(
        _deg_body,
        out_type=jax.ShapeDtypeStruct((NP,), jnp.float32),
        mesh=mesh,
        scratch_types=[
            pltpu.VMEM((CPT_DEG, K), jnp.int32),
            pltpu.VMEM((K,), jnp.float32),
            pltpu.VMEM((ROWS_T,), jnp.float32),
            pltpu.VMEM_SHARED((NP,), jnp.float32),
        ],
    )
    agg_kernel = pl.kernel(
        _agg_body,
        out_type=jax.ShapeDtypeStruct((2, NP, H), jnp.float32),
        mesh=mesh,
        scratch_types=[
            pltpu.VMEM((GCPT, K), jnp.int32),
            pltpu.VMEM((GCPT, K), jnp.int32),
            pltpu.VMEM((K, H), jnp.float32),
            pltpu.VMEM((K, H), jnp.float32),
            pltpu.VMEM((K, H), jnp.float32),
            pltpu.VMEM_SHARED((NP, H), jnp.float32),
            pltpu.SemaphoreType.DMA,
            pltpu.SemaphoreType.DMA,
            pltpu.SemaphoreType.DMA,
            pltpu.SemaphoreType.DMA,
            pltpu.SemaphoreType.DMA,
            pltpu.SemaphoreType.DMA,
        ],
    )
    return deg_kernel, agg_kernel


def _emb_body(x_ref, deg_ref, tcat_ref, w0_ref, p0_ref, dinv_ref):
    m0 = lax.dot_general(tcat_ref[...], w0_ref[...], (((1,), (0,)), ((), ())),
                         preferred_element_type=jnp.float32)
    xt = x_ref[...]
    iota = lax.broadcasted_iota(jnp.int32, (BLK, TPAD), 1)
    cmat = jnp.zeros((BLK, TPAD), jnp.float32)
    off = 0
    for i, d in enumerate(ATOM_DIMS):
        cmat = cmat + jnp.where(iota == xt[:, i:i + 1] + off, 1.0, 0.0)
        off += d
    h = lax.dot_general(cmat, m0, (((1,), (0,)), ((), ())),
                        preferred_element_type=jnp.float32)
    dinv = lax.rsqrt(deg_ref[...] + 1.0)
    p0_ref[...] = dinv * h
    dinv_ref[...] = dinv


_emb_call = pl.pallas_call(
    _emb_body,
    grid=(NP // BLK,),
    in_specs=[
        pl.BlockSpec((BLK, 16), lambda i: (i, 0)),
        pl.BlockSpec((BLK, 1), lambda i: (i, 0)),
        pl.BlockSpec((TPAD, H), lambda i: (0, 0)),
        pl.BlockSpec((H, H), lambda i: (0, 0)),
    ],
    out_specs=[
        pl.BlockSpec((BLK, H), lambda i: (i, 0)),
        pl.BlockSpec((BLK, 1), lambda i: (i, 0)),
    ],
    out_shape=[
        jax.ShapeDtypeStruct((NP, H), jnp.float32),
        jax.ShapeDtypeStruct((NP, 1), jnp.float32),
    ],
)


def _layer_body(acc_ref, p_ref, dinv_ref, b_ref, bng_ref, bnb_ref, w_ref,
                out_ref):
    agg = acc_ref[0] + acc_ref[1] + p_ref[...]
    conv = dinv_ref[...] * agg + b_ref[...]
    h = jnp.maximum(conv * (BN_SCALE * bng_ref[...]) + bnb_ref[...], 0.0)
    out_ref[...] = dinv_ref[...] * lax.dot_general(
        h, w_ref[...], (((1,), (0,)), ((), ())),
        preferred_element_type=jnp.float32)


_layer_call = pl.pallas_call(
    _layer_body,
    grid=(NP // BLK,),
    in_specs=[
        pl.BlockSpec((2, BLK, H), lambda i: (0, i, 0)),
        pl.BlockSpec((BLK, H), lambda i: (i, 0)),
        pl.BlockSpec((BLK, 1), lambda i: (i, 0)),
        pl.BlockSpec((1, H), lambda i: (0, 0)),
        pl.BlockSpec((1, H), lambda i: (0, 0)),
        pl.BlockSpec((1, H), lambda i: (0, 0)),
        pl.BlockSpec((H, H), lambda i: (0, 0)),
    ],
    out_specs=pl.BlockSpec((BLK, H), lambda i: (i, 0)),
    out_shape=jax.ShapeDtypeStruct((NP, H), jnp.float32),
)


def _final_body(acc_ref, p_ref, dinv_ref, b2_ref, batch_ref, linw_ref,
                linb_ref, out_ref, sums_ref, cnts_ref):
    i = pl.program_id(0)

    @pl.when(i == 0)
    def _():
        sums_ref[...] = jnp.zeros_like(sums_ref)
        cnts_ref[...] = jnp.zeros_like(cnts_ref)

    h = dinv_ref[...] * (acc_ref[0] + acc_ref[1] + p_ref[...]) + b2_ref[...]
    ind = jnp.where(
        batch_ref[...] == lax.broadcasted_iota(jnp.int32, (BLK, G), 1),
        1.0, 0.0)
    sums_ref[...] += lax.dot_general(ind, h, (((0,), (0,)), ((), ())),
                                     preferred_element_type=jnp.float32)
    cnts_ref[...] += lax.dot_general(ind, jnp.ones((BLK, 1), jnp.float32),
                                     (((0,), (0,)), ((), ())),
                                     preferred_element_type=jnp.float32)

    @pl.when(i == pl.num_programs(0) - 1)
    def _():
        feats = sums_ref[...] / jnp.maximum(cnts_ref[...], 1.0)
        z = lax.dot_general(feats, linw_ref[...], (((1,), (0,)), ((), ())),
                            preferred_element_type=jnp.float32) + linb_ref[...]
        out_ref[...] = 1.0 / (1.0 + jnp.exp(-z))


_final_call = pl.pallas_call(
    _final_body,
    grid=(NP // BLK,),
    in_specs=[
        pl.BlockSpec((2, BLK, H), lambda i: (0, i, 0)),
        pl.BlockSpec((BLK, H), lambda i: (i, 0)),
        pl.BlockSpec((BLK, 1), lambda i: (i, 0)),
        pl.BlockSpec((1, H), lambda i: (0, 0)),
        pl.BlockSpec((BLK, 1), lambda i: (i, 0)),
        pl.BlockSpec((H, 1), lambda i: (0, 0)),
        pl.BlockSpec((1, 1), lambda i: (0, 0)),
    ],
    out_specs=pl.BlockSpec((G, 1), lambda i: (0, 0)),
    out_shape=jax.ShapeDtypeStruct((G, 1), jnp.float32),
    scratch_shapes=[
        pltpu.VMEM((G, H), jnp.float32),
        pltpu.VMEM((G, 1), jnp.float32),
    ],
)


def kernel(x, edge_index, batch, emb0, emb1, emb2, emb3, emb4, emb5, emb6,
           emb7, emb8, W0, b0, W1, b1, W2, b2, bn0_g, bn0_b, bn1_g, bn1_b,
           lin_W, lin_b):
    xp = jnp.pad(x, ((0, NP - N), (0, 16 - 9)))
    src4d = edge_index[0].reshape(NT, NGRP, GCPT, K)
    dst4d = edge_index[1].reshape(NT, NGRP, GCPT, K)
    dst3d_deg = edge_index[1].reshape(16, CPT_DEG, K)
    batch_p = jnp.pad(batch, (0, NP - N), constant_values=G).reshape(NP, 1)
    tcat = jnp.pad(
        jnp.concatenate([emb0, emb1, emb2, emb3, emb4, emb5, emb6, emb7,
                         emb8], axis=0), ((0, TPAD - TOT), (0, 0)))

    deg_kernel, agg_kernel = _sc_kernels()
    deg = deg_kernel(dst3d_deg).reshape(NP, 1)
    p0, dinv = _emb_call(xp, deg, tcat, W0)
    acc0 = agg_kernel(p0, src4d, dst4d)
    p1 = _layer_call(acc0, p0, dinv, b0.reshape(1, H), bn0_g.reshape(1, H),
                     bn0_b.reshape(1, H), W1)
    acc1 = agg_kernel(p1, src4d, dst4d)
    p2 = _layer_call(acc1, p1, dinv, b1.reshape(1, H), bn1_g.reshape(1, H),
                     bn1_b.reshape(1, H), W2)
    acc2 = agg_kernel(p2, src4d, dst4d)
    out = _final_call(acc2, p2, dinv, b2.reshape(1, H), batch_p, lin_W,
                      lin_b.reshape(1, 1))
    return out


# speedup vs baseline: 26.0866x; 1.0074x over previous
import functools

import numpy as np
import jax
import jax.numpy as jnp
from jax import lax
from jax.experimental import pallas as pl
from jax.experimental.pallas import tpu as pltpu
from jax.experimental.pallas import tpu_sc as plsc

N = 10000
NP = 10240
E = 320000
H = 128
G = 128
K = 100
NT = 32
CPT = E // NT // K
NGRP = 4
GCPT = CPT // NGRP
CPT_DEG = E // NT // K
ROWS_T = NP // 16
BLK = 1024
ATOM_DIMS = [119, 4, 12, 12, 10, 6, 6, 2, 2]
TOT = sum(ATOM_DIMS)
TPAD = 176
BN_SCALE = float(1.0 / np.sqrt(1.0 + 1e-5))

def _deg_body(dst3d, deg0_out, deg1_out, idx_v, ones_v, zero_v, acc_sh):
    c = lax.axis_index("c")
    s = lax.axis_index("s")
    wid = c * 16 + s

    for i in range(7):
        ones_v[pl.ds(i * 16, 16)] = jnp.ones((16,), jnp.float32)
    for i in range(ROWS_T // 16):
        zero_v[pl.ds(i * 16, 16)] = jnp.zeros((16,), jnp.float32)
    pltpu.sync_copy(zero_v, acc_sh.at[pl.ds(s * ROWS_T, ROWS_T)])
    plsc.subcore_barrier()
    pltpu.sync_copy(dst3d.at[wid], idx_v)

    def body(j, carry):
        pltpu.sync_copy(ones_v.at[pl.ds(0, K)], acc_sh.at[idx_v.at[j]],
                        add=True)
        return carry

    lax.fori_loop(0, CPT_DEG, body, 0)
    plsc.subcore_barrier()

    @pl.when(c == 0)
    def _():
        pltpu.sync_copy(acc_sh.at[pl.ds(s * ROWS_T, ROWS_T)],
                        deg0_out.at[pl.ds(s * ROWS_T, ROWS_T)])

    @pl.when(c == 1)
    def _():
        pltpu.sync_copy(acc_sh.at[pl.ds(s * ROWS_T, ROWS_T)],
                        deg1_out.at[pl.ds(s * ROWS_T, ROWS_T)])


def _agg_body(p_hbm, src4d, dst4d, out, sidx, didx, buf0, buf1, buf2,
              acc_sh, gsem0, gsem1, gsem2, ssem0, ssem1, ssem2):
    c = lax.axis_index("c")
    s = lax.axis_index("s")
    wid = c * 16 + s

    def zb(i, carry):
        buf0[i // 8, pl.ds((i % 8) * 16, 16)] = jnp.zeros((16,), jnp.float32)
        return carry

    lax.fori_loop(0, K * 8, zb, 0)
    for kk in range(ROWS_T // 80):
        pltpu.sync_copy(buf0.at[pl.ds(0, 80)],
                        acc_sh.at[pl.ds(s * ROWS_T + kk * 80, 80)])
    plsc.subcore_barrier()

    bufs = (buf0, buf1, buf2)
    gsems = (gsem0, gsem1, gsem2)
    ssems = (ssem0, ssem1, ssem2)
    for g in range(NGRP):
        pltpu.sync_copy(src4d.at[wid, g], sidx)
        pltpu.sync_copy(dst4d.at[wid, g], didx)
        pltpu.async_copy(p_hbm.at[sidx.at[0]], buf0, gsem0)
        pltpu.async_copy(p_hbm.at[sidx.at[1]], buf1, gsem1)

        def outer(jj, carry):
            for b in range(3):
                j = jj * 3 + b

                @pl.when(j < GCPT)
                def _():
                    pltpu.make_async_copy(p_hbm.at[sidx.at[j]], bufs[b],
                                          gsems[b]).wait()
                    pltpu.async_copy(bufs[b], acc_sh.at[didx.at[j]],
                                     ssems[b], add=True)
                    bn = (b + 2) % 3

                    @pl.when(j + 2 < GCPT)
                    def _():
                        @pl.when(j >= 1)
                        def _():
                            pltpu.make_async_copy(
                                bufs[bn], acc_sh.at[didx.at[j]],
                                ssems[bn]).wait()

                        pltpu.async_copy(p_hbm.at[sidx.at[j + 2]],
                                         bufs[bn], gsems[bn])
            return carry

        lax.fori_loop(0, (GCPT + 2) // 3, outer, 0)
        for b in range(3):
            pltpu.make_async_copy(bufs[b], acc_sh.at[didx.at[0]],
                                  ssems[b]).wait()
    plsc.subcore_barrier()
    pltpu.sync_copy(acc_sh.at[pl.ds(s * ROWS_T, ROWS_T)],
                    out.at[c, pl.ds(s * ROWS_T, ROWS_T)])


@functools.cache
def _sc_kernels():
    mesh = plsc.VectorSubcoreMesh(core_axis_name="c", subcore_axis_name="s",
                                  num_cores=2, num_subcores=16)
    deg_kernel = pl.kernel(
        _deg_body,
        out_type=[jax.ShapeDtypeStruct((NP,), jnp.float32),
                  jax.ShapeDtypeStruct((NP,), jnp.float32)],
        mesh=mesh,
        scratch_types=[
            pltpu.VMEM((CPT_DEG, K), jnp.int32),
            pltpu.VMEM((112,), jnp.float32),
            pltpu.VMEM((ROWS_T,), jnp.float32),
            pltpu.VMEM_SHARED((NP,), jnp.float32),
        ],
    )
    agg_kernel = pl.kernel(
        _agg_body,
        out_type=jax.ShapeDtypeStruct((2, NP, H), jnp.float32),
        mesh=mesh,
        scratch_types=[
            pltpu.VMEM((GCPT, K), jnp.int32),
            pltpu.VMEM((GCPT, K), jnp.int32),
            pltpu.VMEM((K, H), jnp.float32),
            pltpu.VMEM((K, H), jnp.float32),
            pltpu.VMEM((K, H), jnp.float32),
            pltpu.VMEM_SHARED((NP, H), jnp.float32),
            pltpu.SemaphoreType.DMA,
            pltpu.SemaphoreType.DMA,
            pltpu.SemaphoreType.DMA,
            pltpu.SemaphoreType.DMA,
            pltpu.SemaphoreType.DMA,
            pltpu.SemaphoreType.DMA,
        ],
    )
    return deg_kernel, agg_kernel


def _emb_body(x_ref, tcat_ref, w0_ref, ht_ref):
    m0 = lax.dot_general(tcat_ref[...], w0_ref[...], (((1,), (0,)), ((), ())),
                         preferred_element_type=jnp.float32)
    xt = x_ref[...]
    iota = lax.broadcasted_iota(jnp.int32, (BLK, TPAD), 1)
    cmat = jnp.zeros((BLK, TPAD), jnp.float32)
    off = 0
    for i, d in enumerate(ATOM_DIMS):
        cmat = cmat + jnp.where(iota == xt[:, i:i + 1] + off, 1.0, 0.0)
        off += d
    ht_ref[...] = lax.dot_general(cmat, m0, (((1,), (0,)), ((), ())),
                                  preferred_element_type=jnp.float32)


_emb_call = pl.pallas_call(
    _emb_body,
    grid=(NP // BLK,),
    in_specs=[
        pl.BlockSpec((BLK, 16), lambda i: (i, 0)),
        pl.BlockSpec((TPAD, H), lambda i: (0, 0)),
        pl.BlockSpec((H, H), lambda i: (0, 0)),
    ],
    out_specs=pl.BlockSpec((BLK, H), lambda i: (i, 0)),
    out_shape=jax.ShapeDtypeStruct((NP, H), jnp.float32),
)


def _scale_body(ht_ref, d0_ref, d1_ref, p0_ref, dinv_ref):
    dinv = lax.rsqrt(d0_ref[...] + d1_ref[...] + 1.0)
    p0_ref[...] = dinv * ht_ref[...]
    dinv_ref[...] = dinv


_scale_call = pl.pallas_call(
    _scale_body,
    grid=(NP // BLK,),
    in_specs=[
        pl.BlockSpec((BLK, H), lambda i: (i, 0)),
        pl.BlockSpec((BLK, 1), lambda i: (i, 0)),
        pl.BlockSpec((BLK, 1), lambda i: (i, 0)),
    ],
    out_specs=[
        pl.BlockSpec((BLK, H), lambda i: (i, 0)),
        pl.BlockSpec((BLK, 1), lambda i: (i, 0)),
    ],
    out_shape=[
        jax.ShapeDtypeStruct((NP, H), jnp.float32),
        jax.ShapeDtypeStruct((NP, 1), jnp.float32),
    ],
)


def _layer_body(acc_ref, p_ref, dinv_ref, b_ref, bng_ref, bnb_ref, w_ref,
                out_ref):
    agg = acc_ref[0] + acc_ref[1] + p_ref[...]
    conv = dinv_ref[...] * agg + b_ref[...]
    h = jnp.maximum(conv * (BN_SCALE * bng_ref[...]) + bnb_ref[...], 0.0)
    out_ref[...] = dinv_ref[...] * lax.dot_general(
        h, w_ref[...], (((1,), (0,)), ((), ())),
        preferred_element_type=jnp.float32)


_layer_call = pl.pallas_call(
    _layer_body,
    grid=(NP // BLK,),
    in_specs=[
        pl.BlockSpec((2, BLK, H), lambda i: (0, i, 0)),
        pl.BlockSpec((BLK, H), lambda i: (i, 0)),
        pl.BlockSpec((BLK, 1), lambda i: (i, 0)),
        pl.BlockSpec((1, H), lambda i: (0, 0)),
        pl.BlockSpec((1, H), lambda i: (0, 0)),
        pl.BlockSpec((1, H), lambda i: (0, 0)),
        pl.BlockSpec((H, H), lambda i: (0, 0)),
    ],
    out_specs=pl.BlockSpec((BLK, H), lambda i: (i, 0)),
    out_shape=jax.ShapeDtypeStruct((NP, H), jnp.float32),
)


def _final_body(acc_ref, p_ref, dinv_ref, b2_ref, batch_ref, linw_ref,
                linb_ref, out_ref, sums_ref, cnts_ref):
    i = pl.program_id(0)

    @pl.when(i == 0)
    def _():
        sums_ref[...] = jnp.zeros_like(sums_ref)
        cnts_ref[...] = jnp.zeros_like(cnts_ref)

    h = dinv_ref[...] * (acc_ref[0] + acc_ref[1] + p_ref[...]) + b2_ref[...]
    ind = jnp.where(
        batch_ref[...] == lax.broadcasted_iota(jnp.int32, (BLK, G), 1),
        1.0, 0.0)
    sums_ref[...] += lax.dot_general(ind, h, (((0,), (0,)), ((), ())),
                                     preferred_element_type=jnp.float32)
    cnts_ref[...] += lax.dot_general(ind, jnp.ones((BLK, 1), jnp.float32),
                                     (((0,), (0,)), ((), ())),
                                     preferred_element_type=jnp.float32)

    @pl.when(i == pl.num_programs(0) - 1)
    def _():
        feats = sums_ref[...] / jnp.maximum(cnts_ref[...], 1.0)
        z = lax.dot_general(feats, linw_ref[...], (((1,), (0,)), ((), ())),
                            preferred_element_type=jnp.float32) + linb_ref[...]
        out_ref[...] = 1.0 / (1.0 + jnp.exp(-z))


_final_call = pl.pallas_call(
    _final_body,
    grid=(NP // BLK,),
    in_specs=[
        pl.BlockSpec((2, BLK, H), lambda i: (0, i, 0)),
        pl.BlockSpec((BLK, H), lambda i: (i, 0)),
        pl.BlockSpec((BLK, 1), lambda i: (i, 0)),
        pl.BlockSpec((1, H), lambda i: (0, 0)),
        pl.BlockSpec((BLK, 1), lambda i: (i, 0)),
        pl.BlockSpec((H, 1), lambda i: (0, 0)),
        pl.BlockSpec((1, 1), lambda i: (0, 0)),
    ],
    out_specs=pl.BlockSpec((G, 1), lambda i: (0, 0)),
    out_shape=jax.ShapeDtypeStruct((G, 1), jnp.float32),
    scratch_shapes=[
        pltpu.VMEM((G, H), jnp.float32),
        pltpu.VMEM((G, 1), jnp.float32),
    ],
)


def kernel(x, edge_index, batch, emb0, emb1, emb2, emb3, emb4, emb5, emb6,
           emb7, emb8, W0, b0, W1, b1, W2, b2, bn0_g, bn0_b, bn1_g, bn1_b,
           lin_W, lin_b):
    xp = jnp.pad(x, ((0, NP - N), (0, 16 - 9)))
    src4d = edge_index[0].reshape(NT, NGRP, GCPT, K)
    dst4d = edge_index[1].reshape(NT, NGRP, GCPT, K)
    dst3d_deg = edge_index[1].reshape(NT, CPT_DEG, K)
    batch_p = jnp.pad(batch, (0, NP - N), constant_values=G).reshape(NP, 1)
    tcat = jnp.pad(
        jnp.concatenate([emb0, emb1, emb2, emb3, emb4, emb5, emb6, emb7,
                         emb8], axis=0), ((0, TPAD - TOT), (0, 0)))

    deg_kernel, agg_kernel = _sc_kernels()
    deg0, deg1 = deg_kernel(dst3d_deg)
    ht = _emb_call(xp, tcat, W0)
    p0, dinv = _scale_call(ht, deg0.reshape(NP, 1), deg1.reshape(NP, 1))
    acc0 = agg_kernel(p0, src4d, dst4d)
    p1 = _layer_call(acc0, p0, dinv, b0.reshape(1, H), bn0_g.reshape(1, H),
                     bn0_b.reshape(1, H), W1)
    acc1 = agg_kernel(p1, src4d, dst4d)
    p2 = _layer_call(acc1, p1, dinv, b1.reshape(1, H), bn1_g.reshape(1, H),
                     bn1_b.reshape(1, H), W2)
    acc2 = agg_kernel(p2, src4d, dst4d)
    out = _final_call(acc2, p2, dinv, b2.reshape(1, H), batch_p, lin_W,
                      lin_b.reshape(1, 1))
    return out


# speedup vs baseline: 26.2897x; 1.0078x over previous
import functools

import numpy as np
import jax
import jax.numpy as jnp
from jax import lax
from jax.experimental import pallas as pl
from jax.experimental.pallas import tpu as pltpu
from jax.experimental.pallas import tpu_sc as plsc

N = 10000
NP = 10240
E = 320000
H = 128
G = 128
K = 80
NT = 32
CPT = E // NT // K
NGRP = 5
GCPT = CPT // NGRP
CPT_DEG = E // NT // K
ROWS_T = NP // 16
BLK = 1024
ATOM_DIMS = [119, 4, 12, 12, 10, 6, 6, 2, 2]
TOT = sum(ATOM_DIMS)
TPAD = 176
BN_SCALE = float(1.0 / np.sqrt(1.0 + 1e-5))

def _deg_body(dst3d, deg0_out, deg1_out, idx_v, ones_v, zero_v, acc_sh):
    c = lax.axis_index("c")
    s = lax.axis_index("s")
    wid = c * 16 + s

    for i in range(7):
        ones_v[pl.ds(i * 16, 16)] = jnp.ones((16,), jnp.float32)
    for i in range(ROWS_T // 16):
        zero_v[pl.ds(i * 16, 16)] = jnp.zeros((16,), jnp.float32)
    pltpu.sync_copy(zero_v, acc_sh.at[pl.ds(s * ROWS_T, ROWS_T)])
    plsc.subcore_barrier()
    pltpu.sync_copy(dst3d.at[wid], idx_v)

    def body(j, carry):
        pltpu.sync_copy(ones_v.at[pl.ds(0, K)], acc_sh.at[idx_v.at[j]],
                        add=True)
        return carry

    lax.fori_loop(0, CPT_DEG, body, 0)
    plsc.subcore_barrier()

    @pl.when(c == 0)
    def _():
        pltpu.sync_copy(acc_sh.at[pl.ds(s * ROWS_T, ROWS_T)],
                        deg0_out.at[pl.ds(s * ROWS_T, ROWS_T)])

    @pl.when(c == 1)
    def _():
        pltpu.sync_copy(acc_sh.at[pl.ds(s * ROWS_T, ROWS_T)],
                        deg1_out.at[pl.ds(s * ROWS_T, ROWS_T)])


def _agg_body(p_hbm, src4d, dst4d, out, cidx, buf0, buf1, buf2, buf3,
              acc_sh, gsem0, gsem1, gsem2, gsem3, ssem0, ssem1, ssem2, ssem3):
    c = lax.axis_index("c")
    s = lax.axis_index("s")
    wid = c * 16 + s

    def zb(i, carry):
        buf0[i // 8, pl.ds((i % 8) * 16, 16)] = jnp.zeros((16,), jnp.float32)
        return carry

    lax.fori_loop(0, K * 8, zb, 0)
    for kk in range(ROWS_T // K):
        pltpu.sync_copy(buf0, acc_sh.at[pl.ds(s * ROWS_T + kk * K, K)])
    plsc.subcore_barrier()

    bufs = (buf0, buf1, buf2, buf3)
    gsems = (gsem0, gsem1, gsem2, gsem3)
    ssems = (ssem0, ssem1, ssem2, ssem3)
    for g in range(NGRP):
        pltpu.sync_copy(src4d.at[wid, g], cidx.at[pl.ds(0, GCPT)])
        pltpu.sync_copy(dst4d.at[wid, g], cidx.at[pl.ds(GCPT, GCPT)])
        pltpu.async_copy(p_hbm.at[cidx.at[0]], buf0, gsem0)
        pltpu.async_copy(p_hbm.at[cidx.at[1]], buf1, gsem1)
        pltpu.async_copy(p_hbm.at[cidx.at[2]], buf2, gsem2)

        def outer(jj, carry):
            for b in range(4):
                j = jj * 4 + b

                @pl.when(j < GCPT)
                def _():
                    pltpu.make_async_copy(p_hbm.at[cidx.at[j]], bufs[b],
                                          gsems[b]).wait()
                    pltpu.async_copy(bufs[b], acc_sh.at[cidx.at[GCPT + j]],
                                     ssems[b], add=True)
                    bn = (b + 3) % 4

                    @pl.when(j + 3 < GCPT)
                    def _():
                        @pl.when(j >= 1)
                        def _():
                            pltpu.make_async_copy(
                                bufs[bn], acc_sh.at[cidx.at[GCPT]],
                                ssems[bn]).wait()

                        pltpu.async_copy(p_hbm.at[cidx.at[j + 3]],
                                         bufs[bn], gsems[bn])
            return carry

        lax.fori_loop(0, (GCPT + 3) // 4, outer, 0)
        for b in range(4):
            pltpu.make_async_copy(bufs[b], acc_sh.at[cidx.at[GCPT]],
                                  ssems[b]).wait()
    plsc.subcore_barrier()
    pltpu.sync_copy(acc_sh.at[pl.ds(s * ROWS_T, ROWS_T)],
                    out.at[c, pl.ds(s * ROWS_T, ROWS_T)])


@functools.cache
def _sc_kernels():
    mesh = plsc.VectorSubcoreMesh(core_axis_name="c", subcore_axis_name="s",
                                  num_cores=2, num_subcores=16)
    deg_kernel = pl.kernel(
        _deg_body,
        out_type=[jax.ShapeDtypeStruct((NP,), jnp.float32),
                  jax.ShapeDtypeStruct((NP,), jnp.float32)],
        mesh=mesh,
        scratch_types=[
            pltpu.VMEM((CPT_DEG, K), jnp.int32),
            pltpu.VMEM((112,), jnp.float32),
            pltpu.VMEM((ROWS_T,), jnp.float32),
            pltpu.VMEM_SHARED((NP,), jnp.float32),
        ],
    )
    agg_kernel = pl.kernel(
        _agg_body,
        out_type=jax.ShapeDtypeStruct((2, NP, H), jnp.float32),
        mesh=mesh,
        scratch_types=[
            pltpu.VMEM((2 * GCPT, K), jnp.int32),
            pltpu.VMEM((K, H), jnp.float32),
            pltpu.VMEM((K, H), jnp.float32),
            pltpu.VMEM((K, H), jnp.float32),
            pltpu.VMEM((K, H), jnp.float32),
            pltpu.VMEM_SHARED((NP, H), jnp.float32),
            pltpu.SemaphoreType.DMA,
            pltpu.SemaphoreType.DMA,
            pltpu.SemaphoreType.DMA,
            pltpu.SemaphoreType.DMA,
            pltpu.SemaphoreType.DMA,
            pltpu.SemaphoreType.DMA,
            pltpu.SemaphoreType.DMA,
            pltpu.SemaphoreType.DMA,
        ],
    )
    return deg_kernel, agg_kernel


def _emb_body(x_ref, d0_ref, d1_ref, tcat_ref, w0_ref, p0_ref, dinv_ref):
    m0 = lax.dot_general(tcat_ref[...], w0_ref[...], (((1,), (0,)), ((), ())),
                         preferred_element_type=jnp.float32)
    xt = x_ref[...]
    iota = lax.broadcasted_iota(jnp.int32, (BLK, TPAD), 1)
    cmat = jnp.zeros((BLK, TPAD), jnp.float32)
    off = 0
    for i, d in enumerate(ATOM_DIMS):
        cmat = cmat + jnp.where(iota == xt[:, i:i + 1] + off, 1.0, 0.0)
        off += d
    h = lax.dot_general(cmat, m0, (((1,), (0,)), ((), ())),
                        preferred_element_type=jnp.float32)
    dinv = lax.rsqrt(d0_ref[...] + d1_ref[...] + 1.0).reshape(BLK, 1)
    p0_ref[...] = dinv * h
    dinv_ref[...] = dinv.reshape(BLK)


_emb_call = pl.pallas_call(
    _emb_body,
    grid=(NP // BLK,),
    in_specs=[
        pl.BlockSpec((BLK, 9), lambda i: (i, 0)),
        pl.BlockSpec((BLK,), lambda i: (i,)),
        pl.BlockSpec((BLK,), lambda i: (i,)),
        pl.BlockSpec((TPAD, H), lambda i: (0, 0)),
        pl.BlockSpec((H, H), lambda i: (0, 0)),
    ],
    out_specs=[
        pl.BlockSpec((BLK, H), lambda i: (i, 0)),
        pl.BlockSpec((BLK,), lambda i: (i,)),
    ],
    out_shape=[
        jax.ShapeDtypeStruct((NP, H), jnp.float32),
        jax.ShapeDtypeStruct((NP,), jnp.float32),
    ],
)


def _layer_body(acc_ref, p_ref, dinv_ref, b_ref, bng_ref, bnb_ref, w_ref,
                out_ref):
    dinv = dinv_ref[...].reshape(BLK, 1)
    agg = acc_ref[0] + acc_ref[1] + p_ref[...]
    conv = dinv * agg + b_ref[...]
    h = jnp.maximum(conv * (BN_SCALE * bng_ref[...]) + bnb_ref[...], 0.0)
    out_ref[...] = dinv * lax.dot_general(
        h, w_ref[...], (((1,), (0,)), ((), ())),
        preferred_element_type=jnp.float32)


_layer_call = pl.pallas_call(
    _layer_body,
    grid=(NP // BLK,),
    in_specs=[
        pl.BlockSpec((2, BLK, H), lambda i: (0, i, 0)),
        pl.BlockSpec((BLK, H), lambda i: (i, 0)),
        pl.BlockSpec((BLK,), lambda i: (i,)),
        pl.BlockSpec((1, H), lambda i: (0, 0)),
        pl.BlockSpec((1, H), lambda i: (0, 0)),
        pl.BlockSpec((1, H), lambda i: (0, 0)),
        pl.BlockSpec((H, H), lambda i: (0, 0)),
    ],
    out_specs=pl.BlockSpec((BLK, H), lambda i: (i, 0)),
    out_shape=jax.ShapeDtypeStruct((NP, H), jnp.float32),
)


def _final_body(acc_ref, p_ref, dinv_ref, b2_ref, batch_ref, linw_ref,
                linb_ref, out_ref, sums_ref, cnts_ref):
    i = pl.program_id(0)

    @pl.when(i == 0)
    def _():
        sums_ref[...] = jnp.zeros_like(sums_ref)
        cnts_ref[...] = jnp.zeros_like(cnts_ref)

    dinv = dinv_ref[...].reshape(BLK, 1)
    h = dinv * (acc_ref[0] + acc_ref[1] + p_ref[...]) + b2_ref[...]
    ind = jnp.where(
        batch_ref[...].reshape(BLK, 1)
        == lax.broadcasted_iota(jnp.int32, (BLK, G), 1), 1.0, 0.0)
    sums_ref[...] += lax.dot_general(ind, h, (((0,), (0,)), ((), ())),
                                     preferred_element_type=jnp.float32)
    cnts_ref[...] += lax.dot_general(ind, jnp.ones((BLK, 1), jnp.float32),
                                     (((0,), (0,)), ((), ())),
                                     preferred_element_type=jnp.float32)

    @pl.when(i == pl.num_programs(0) - 1)
    def _():
        feats = sums_ref[...] / jnp.maximum(cnts_ref[...], 1.0)
        z = lax.dot_general(feats, linw_ref[...], (((1,), (0,)), ((), ())),
                            preferred_element_type=jnp.float32) + linb_ref[...]
        out_ref[...] = 1.0 / (1.0 + jnp.exp(-z))


_final_call = pl.pallas_call(
    _final_body,
    grid=(NP // BLK,),
    in_specs=[
        pl.BlockSpec((2, BLK, H), lambda i: (0, i, 0)),
        pl.BlockSpec((BLK, H), lambda i: (i, 0)),
        pl.BlockSpec((BLK,), lambda i: (i,)),
        pl.BlockSpec((1, H), lambda i: (0, 0)),
        pl.BlockSpec((BLK,), lambda i: (i,)),
        pl.BlockSpec((H, 1), lambda i: (0, 0)),
        pl.BlockSpec((1, 1), lambda i: (0, 0)),
    ],
    out_specs=pl.BlockSpec((G, 1), lambda i: (0, 0)),
    out_shape=jax.ShapeDtypeStruct((G, 1), jnp.float32),
    scratch_shapes=[
        pltpu.VMEM((G, H), jnp.float32),
        pltpu.VMEM((G, 1), jnp.float32),
    ],
)


def kernel(x, edge_index, batch, emb0, emb1, emb2, emb3, emb4, emb5, emb6,
           emb7, emb8, W0, b0, W1, b1, W2, b2, bn0_g, bn0_b, bn1_g, bn1_b,
           lin_W, lin_b):
    deg_kernel, agg_kernel = _sc_kernels()
    dst3d_deg = edge_index[1].reshape(NT, CPT_DEG, K)
    deg0, deg1 = deg_kernel(dst3d_deg)
    src4d = edge_index[0].reshape(NT, NGRP, GCPT, K)
    dst4d = edge_index[1].reshape(NT, NGRP, GCPT, K)
    batch_p = jnp.pad(batch, (0, NP - N), constant_values=G)
    tcat = jnp.pad(
        jnp.concatenate([emb0, emb1, emb2, emb3, emb4, emb5, emb6, emb7,
                         emb8], axis=0), ((0, TPAD - TOT), (0, 0)))

    p0, dinv = _emb_call(x, deg0, deg1, tcat, W0)
    acc0 = agg_kernel(p0, src4d, dst4d)
    p1 = _layer_call(acc0, p0, dinv, b0.reshape(1, H), bn0_g.reshape(1, H),
                     bn0_b.reshape(1, H), W1)
    acc1 = agg_kernel(p1, src4d, dst4d)
    p2 = _layer_call(acc1, p1, dinv, b1.reshape(1, H), bn1_g.reshape(1, H),
                     bn1_b.reshape(1, H), W2)
    acc2 = agg_kernel(p2, src4d, dst4d)
    out = _final_call(acc2, p2, dinv, b2.reshape(1, H), batch_p, lin_W,
                      lin_b.reshape(1, 1))
    return out


# speedup vs baseline: 27.2802x; 1.0377x over previous
import functools

import numpy as np
import jax
import jax.numpy as jnp
from jax import lax
from jax.experimental import pallas as pl
from jax.experimental.pallas import tpu as pltpu
from jax.experimental.pallas import tpu_sc as plsc

N = 10000
NP = 10240
E = 320000
H = 128
G = 128
K = 100
NT = 32
CPT = E // NT // K
NGRP = 4
GCPT = CPT // NGRP
CPT_DEG = E // NT // K
ROWS_T = NP // 16
BLK = 1024
ATOM_DIMS = [119, 4, 12, 12, 10, 6, 6, 2, 2]
TOT = sum(ATOM_DIMS)
TPAD = 176
BN_SCALE = float(1.0 / np.sqrt(1.0 + 1e-5))

def _deg_body(dst3d, deg0_out, deg1_out, idx_v, ones_v, zero_v, acc_sh):
    c = lax.axis_index("c")
    s = lax.axis_index("s")
    wid = c * 16 + s

    for i in range(7):
        ones_v[pl.ds(i * 16, 16)] = jnp.ones((16,), jnp.float32)
    for i in range(ROWS_T // 16):
        zero_v[pl.ds(i * 16, 16)] = jnp.zeros((16,), jnp.float32)
    pltpu.sync_copy(zero_v, acc_sh.at[pl.ds(s * ROWS_T, ROWS_T)])
    plsc.subcore_barrier()
    pltpu.sync_copy(dst3d.at[wid], idx_v)

    def body(j, carry):
        pltpu.sync_copy(ones_v.at[pl.ds(0, K)], acc_sh.at[idx_v.at[j]],
                        add=True)
        return carry

    lax.fori_loop(0, CPT_DEG, body, 0)
    plsc.subcore_barrier()

    @pl.when(c == 0)
    def _():
        pltpu.sync_copy(acc_sh.at[pl.ds(s * ROWS_T, ROWS_T)],
                        deg0_out.at[pl.ds(s * ROWS_T, ROWS_T)])

    @pl.when(c == 1)
    def _():
        pltpu.sync_copy(acc_sh.at[pl.ds(s * ROWS_T, ROWS_T)],
                        deg1_out.at[pl.ds(s * ROWS_T, ROWS_T)])


def _agg_body(p_hbm, src4d, dst4d, out, sidx, didx, buf0, buf1, buf2,
              acc_sh, gsem0, gsem1, gsem2, hsem0, hsem1, hsem2,
              ssem0, ssem1, ssem2):
    c = lax.axis_index("c")
    s = lax.axis_index("s")
    wid = c * 16 + s

    def zb(i, carry):
        buf0[i // 8, pl.ds((i % 8) * 16, 16)] = jnp.zeros((16,), jnp.float32)
        return carry

    lax.fori_loop(0, K * 8, zb, 0)
    for kk in range(ROWS_T // 80):
        pltpu.sync_copy(buf0.at[pl.ds(0, 80)],
                        acc_sh.at[pl.ds(s * ROWS_T + kk * 80, 80)])
    plsc.subcore_barrier()

    bufs = (buf0, buf1, buf2)
    gsems = (gsem0, gsem1, gsem2)
    hsems = (hsem0, hsem1, hsem2)
    ssems = (ssem0, ssem1, ssem2)
    KH = K // 2

    def start_gather(j, b):
        pltpu.async_copy(p_hbm.at[sidx.at[j, pl.ds(0, KH)]],
                         bufs[b].at[pl.ds(0, KH)], gsems[b])
        pltpu.async_copy(p_hbm.at[sidx.at[j, pl.ds(KH, KH)]],
                         bufs[b].at[pl.ds(KH, KH)], hsems[b])

    def wait_gather(j, b):
        pltpu.make_async_copy(p_hbm.at[sidx.at[j, pl.ds(0, KH)]],
                              bufs[b].at[pl.ds(0, KH)], gsems[b]).wait()
        pltpu.make_async_copy(p_hbm.at[sidx.at[j, pl.ds(KH, KH)]],
                              bufs[b].at[pl.ds(KH, KH)], hsems[b]).wait()

    for g in range(NGRP):
        pltpu.sync_copy(src4d.at[wid, g], sidx)
        pltpu.sync_copy(dst4d.at[wid, g], didx)
        start_gather(0, 0)
        start_gather(1, 1)

        def outer(jj, carry):
            for b in range(3):
                j = jj * 3 + b

                @pl.when(j < GCPT)
                def _():
                    wait_gather(j, b)
                    pltpu.async_copy(bufs[b], acc_sh.at[didx.at[j]],
                                     ssems[b], add=True)
                    bn = (b + 2) % 3

                    @pl.when(j + 2 < GCPT)
                    def _():
                        @pl.when(j >= 1)
                        def _():
                            pltpu.make_async_copy(
                                bufs[bn], acc_sh.at[didx.at[j]],
                                ssems[bn]).wait()

                        start_gather(j + 2, bn)
            return carry

        lax.fori_loop(0, (GCPT + 2) // 3, outer, 0)
        for b in range(3):
            pltpu.make_async_copy(bufs[b], acc_sh.at[didx.at[0]],
                                  ssems[b]).wait()
    plsc.subcore_barrier()
    pltpu.sync_copy(acc_sh.at[pl.ds(s * ROWS_T, ROWS_T)],
                    out.at[c, pl.ds(s * ROWS_T, ROWS_T)])


@functools.cache
def _sc_kernels():
    mesh = plsc.VectorSubcoreMesh(core_axis_name="c", subcore_axis_name="s",
                                  num_cores=2, num_subcores=16)
    deg_kernel = pl.kernel(
        _deg_body,
        out_type=[jax.ShapeDtypeStruct((NP,), jnp.float32),
                  jax.ShapeDtypeStruct((NP,), jnp.float32)],
        mesh=mesh,
        scratch_types=[
            pltpu.VMEM((CPT_DEG, K), jnp.int32),
            pltpu.VMEM((112,), jnp.float32),
            pltpu.VMEM((ROWS_T,), jnp.float32),
            pltpu.VMEM_SHARED((NP,), jnp.float32),
        ],
    )
    agg_kernel = pl.kernel(
        _agg_body,
        out_type=jax.ShapeDtypeStruct((2, NP, H), jnp.float32),
        mesh=mesh,
        scratch_types=[
            pltpu.VMEM((GCPT, K), jnp.int32),
            pltpu.VMEM((GCPT, K), jnp.int32),
            pltpu.VMEM((K, H), jnp.float32),
            pltpu.VMEM((K, H), jnp.float32),
            pltpu.VMEM((K, H), jnp.float32),
            pltpu.VMEM_SHARED((NP, H), jnp.float32),
            pltpu.SemaphoreType.DMA,
            pltpu.SemaphoreType.DMA,
            pltpu.SemaphoreType.DMA,
            pltpu.SemaphoreType.DMA,
            pltpu.SemaphoreType.DMA,
            pltpu.SemaphoreType.DMA,
            pltpu.SemaphoreType.DMA,
            pltpu.SemaphoreType.DMA,
            pltpu.SemaphoreType.DMA,
        ],
    )
    return deg_kernel, agg_kernel


def _emb_body(x_ref, d0_ref, d1_ref, tcat_ref, w0_ref, p0_ref, dinv_ref):
    m0 = lax.dot_general(tcat_ref[...], w0_ref[...], (((1,), (0,)), ((), ())),
                         preferred_element_type=jnp.float32)
    xt = x_ref[...]
    iota = lax.broadcasted_iota(jnp.int32, (BLK, TPAD), 1)
    cmat = jnp.zeros((BLK, TPAD), jnp.float32)
    off = 0
    for i, d in enumerate(ATOM_DIMS):
        cmat = cmat + jnp.where(iota == xt[:, i:i + 1] + off, 1.0, 0.0)
        off += d
    h = lax.dot_general(cmat, m0, (((1,), (0,)), ((), ())),
                        preferred_element_type=jnp.float32)
    dinv = lax.rsqrt(d0_ref[...] + d1_ref[...] + 1.0).reshape(BLK, 1)
    p0_ref[...] = dinv * h
    dinv_ref[...] = dinv.reshape(BLK)


_emb_call = pl.pallas_call(
    _emb_body,
    grid=(NP // BLK,),
    in_specs=[
        pl.BlockSpec((BLK, 9), lambda i: (i, 0)),
        pl.BlockSpec((BLK,), lambda i: (i,)),
        pl.BlockSpec((BLK,), lambda i: (i,)),
        pl.BlockSpec((TPAD, H), lambda i: (0, 0)),
        pl.BlockSpec((H, H), lambda i: (0, 0)),
    ],
    out_specs=[
        pl.BlockSpec((BLK, H), lambda i: (i, 0)),
        pl.BlockSpec((BLK,), lambda i: (i,)),
    ],
    out_shape=[
        jax.ShapeDtypeStruct((NP, H), jnp.float32),
        jax.ShapeDtypeStruct((NP,), jnp.float32),
    ],
)


def _layer_body(acc_ref, p_ref, dinv_ref, b_ref, bng_ref, bnb_ref, w_ref,
                out_ref):
    dinv = dinv_ref[...].reshape(BLK, 1)
    agg = acc_ref[0] + acc_ref[1] + p_ref[...]
    conv = dinv * agg + b_ref[...]
    h = jnp.maximum(conv * (BN_SCALE * bng_ref[...]) + bnb_ref[...], 0.0)
    out_ref[...] = dinv * lax.dot_general(
        h, w_ref[...], (((1,), (0,)), ((), ())),
        preferred_element_type=jnp.float32)


_layer_call = pl.pallas_call(
    _layer_body,
    grid=(NP // BLK,),
    in_specs=[
        pl.BlockSpec((2, BLK, H), lambda i: (0, i, 0)),
        pl.BlockSpec((BLK, H), lambda i: (i, 0)),
        pl.BlockSpec((BLK,), lambda i: (i,)),
        pl.BlockSpec((1, H), lambda i: (0, 0)),
        pl.BlockSpec((1, H), lambda i: (0, 0)),
        pl.BlockSpec((1, H), lambda i: (0, 0)),
        pl.BlockSpec((H, H), lambda i: (0, 0)),
    ],
    out_specs=pl.BlockSpec((BLK, H), lambda i: (i, 0)),
    out_shape=jax.ShapeDtypeStruct((NP, H), jnp.float32),
)


def _final_body(acc_ref, p_ref, dinv_ref, b2_ref, batch_ref, linw_ref,
                linb_ref, out_ref, sums_ref, cnts_ref):
    i = pl.program_id(0)

    @pl.when(i == 0)
    def _():
        sums_ref[...] = jnp.zeros_like(sums_ref)
        cnts_ref[...] = jnp.zeros_like(cnts_ref)

    dinv = dinv_ref[...].reshape(BLK, 1)
    h = dinv * (acc_ref[0] + acc_ref[1] + p_ref[...]) + b2_ref[...]
    ind = jnp.where(
        batch_ref[...].reshape(BLK, 1)
        == lax.broadcasted_iota(jnp.int32, (BLK, G), 1), 1.0, 0.0)
    sums_ref[...] += lax.dot_general(ind, h, (((0,), (0,)), ((), ())),
                                     preferred_element_type=jnp.float32)
    cnts_ref[...] += lax.dot_general(ind, jnp.ones((BLK, 1), jnp.float32),
                                     (((0,), (0,)), ((), ())),
                                     preferred_element_type=jnp.float32)

    @pl.when(i == pl.num_programs(0) - 1)
    def _():
        feats = sums_ref[...] / jnp.maximum(cnts_ref[...], 1.0)
        z = lax.dot_general(feats, linw_ref[...], (((1,), (0,)), ((), ())),
                            preferred_element_type=jnp.float32) + linb_ref[...]
        out_ref[...] = 1.0 / (1.0 + jnp.exp(-z))


_final_call = pl.pallas_call(
    _final_body,
    grid=(NP // BLK,),
    in_specs=[
        pl.BlockSpec((2, BLK, H), lambda i: (0, i, 0)),
        pl.BlockSpec((BLK, H), lambda i: (i, 0)),
        pl.BlockSpec((BLK,), lambda i: (i,)),
        pl.BlockSpec((1, H), lambda i: (0, 0)),
        pl.BlockSpec((BLK,), lambda i: (i,)),
        pl.BlockSpec((H, 1), lambda i: (0, 0)),
        pl.BlockSpec((1, 1), lambda i: (0, 0)),
    ],
    out_specs=pl.BlockSpec((G, 1), lambda i: (0, 0)),
    out_shape=jax.ShapeDtypeStruct((G, 1), jnp.float32),
    scratch_shapes=[
        pltpu.VMEM((G, H), jnp.float32),
        pltpu.VMEM((G, 1), jnp.float32),
    ],
)


def kernel(x, edge_index, batch, emb0, emb1, emb2, emb3, emb4, emb5, emb6,
           emb7, emb8, W0, b0, W1, b1, W2, b2, bn0_g, bn0_b, bn1_g, bn1_b,
           lin_W, lin_b):
    deg_kernel, agg_kernel = _sc_kernels()
    dst3d_deg = edge_index[1].reshape(NT, CPT_DEG, K)
    deg0, deg1 = deg_kernel(dst3d_deg)
    src4d = edge_index[0].reshape(NT, NGRP, GCPT, K)
    dst4d = edge_index[1].reshape(NT, NGRP, GCPT, K)
    batch_p = jnp.pad(batch, (0, NP - N), constant_values=G)
    tcat = jnp.pad(
        jnp.concatenate([emb0, emb1, emb2, emb3, emb4, emb5, emb6, emb7,
                         emb8], axis=0), ((0, TPAD - TOT), (0, 0)))

    p0, dinv = _emb_call(x, deg0, deg1, tcat, W0)
    acc0 = agg_kernel(p0, src4d, dst4d)
    p1 = _layer_call(acc0, p0, dinv, b0.reshape(1, H), bn0_g.reshape(1, H),
                     bn0_b.reshape(1, H), W1)
    acc1 = agg_kernel(p1, src4d, dst4d)
    p2 = _layer_call(acc1, p1, dinv, b1.reshape(1, H), bn1_g.reshape(1, H),
                     bn1_b.reshape(1, H), W2)
    acc2 = agg_kernel(p2, src4d, dst4d)
    out = _final_call(acc2, p2, dinv, b2.reshape(1, H), batch_p, lin_W,
                      lin_b.reshape(1, 1))
    return out


# speedup vs baseline: 27.5405x; 1.0095x over previous
import functools

import numpy as np
import jax
import jax.numpy as jnp
from jax import lax
from jax.experimental import pallas as pl
from jax.experimental.pallas import tpu as pltpu
from jax.experimental.pallas import tpu_sc as plsc

N = 10000
NP = 10240
E = 320000
H = 128
G = 128
K = 100
NT = 32
CPT = E // NT // K
NGRP = 4
GCPT = CPT // NGRP
CPT_DEG = E // NT // K
ROWS_T = NP // 16
BLK = 1024
ATOM_DIMS = [119, 4, 12, 12, 10, 6, 6, 2, 2]
TOT = sum(ATOM_DIMS)
TPAD = 176
BN_SCALE = float(1.0 / np.sqrt(1.0 + 1e-5))

def _deg_body(dst3d, deg0_out, deg1_out, idx_v, ones_v, zero_v, acc_sh):
    c = lax.axis_index("c")
    s = lax.axis_index("s")
    wid = c * 16 + s

    for i in range(7):
        ones_v[pl.ds(i * 16, 16)] = jnp.ones((16,), jnp.float32)
    for i in range(ROWS_T // 16):
        zero_v[pl.ds(i * 16, 16)] = jnp.zeros((16,), jnp.float32)
    pltpu.sync_copy(zero_v, acc_sh.at[pl.ds(s * ROWS_T, ROWS_T)])
    plsc.subcore_barrier()
    pltpu.sync_copy(dst3d.at[wid], idx_v)

    def body(j, carry):
        pltpu.sync_copy(ones_v.at[pl.ds(0, K)], acc_sh.at[idx_v.at[j]],
                        add=True)
        return carry

    lax.fori_loop(0, CPT_DEG, body, 0)
    plsc.subcore_barrier()

    @pl.when(c == 0)
    def _():
        pltpu.sync_copy(acc_sh.at[pl.ds(s * ROWS_T, ROWS_T)],
                        deg0_out.at[pl.ds(s * ROWS_T, ROWS_T)])

    @pl.when(c == 1)
    def _():
        pltpu.sync_copy(acc_sh.at[pl.ds(s * ROWS_T, ROWS_T)],
                        deg1_out.at[pl.ds(s * ROWS_T, ROWS_T)])


def _agg_body(p_hbm, src4d, dst4d, out, sidx, didx, buf0, buf1, buf2,
              acc_sh, gsem0, gsem1, gsem2, ssem0, ssem1, ssem2):
    c = lax.axis_index("c")
    s = lax.axis_index("s")
    wid = c * 16 + s

    def zb(i, carry):
        buf0[i // 8, pl.ds((i % 8) * 16, 16)] = jnp.zeros((16,), jnp.float32)
        return carry

    lax.fori_loop(0, K * 8, zb, 0)
    for kk in range(ROWS_T // 80):
        pltpu.sync_copy(buf0.at[pl.ds(0, 80)],
                        acc_sh.at[pl.ds(s * ROWS_T + kk * 80, 80)])
    plsc.subcore_barrier()

    bufs = (buf0, buf1, buf2)
    gsems = (gsem0, gsem1, gsem2)
    ssems = (ssem0, ssem1, ssem2)
    for g in range(NGRP):
        pltpu.sync_copy(src4d.at[wid, g], sidx)
        pltpu.sync_copy(dst4d.at[wid, g], didx)
        pltpu.async_copy(p_hbm.at[sidx.at[0]], buf0, gsem0)
        pltpu.async_copy(p_hbm.at[sidx.at[1]], buf1, gsem1)

        def outer(jj, carry):
            for b in range(3):
                j = jj * 3 + b

                @pl.when(j < GCPT)
                def _():
                    pltpu.make_async_copy(p_hbm.at[sidx.at[j]], bufs[b],
                                          gsems[b]).wait()
                    pltpu.async_copy(bufs[b], acc_sh.at[didx.at[j]],
                                     ssems[b], add=True)
                    bn = (b + 2) % 3

                    @pl.when(j + 2 < GCPT)
                    def _():
                        @pl.when(j >= 1)
                        def _():
                            pltpu.make_async_copy(
                                bufs[bn], acc_sh.at[didx.at[j]],
                                ssems[bn]).wait()

                        pltpu.async_copy(p_hbm.at[sidx.at[j + 2]],
                                         bufs[bn], gsems[bn])
            return carry

        lax.fori_loop(0, (GCPT + 2) // 3, outer, 0)
        for b in range(3):
            pltpu.make_async_copy(bufs[b], acc_sh.at[didx.at[0]],
                                  ssems[b]).wait()
    plsc.subcore_barrier()
    pltpu.sync_copy(acc_sh.at[pl.ds(s * ROWS_T, ROWS_T)],
                    out.at[c, pl.ds(s * ROWS_T, ROWS_T)])


@functools.cache
def _sc_kernels():
    mesh = plsc.VectorSubcoreMesh(core_axis_name="c", subcore_axis_name="s",
                                  num_cores=2, num_subcores=16)
    deg_kernel = pl.kernel(
        _deg_body,
        out_type=[jax.ShapeDtypeStruct((NP,), jnp.float32),
                  jax.ShapeDtypeStruct((NP,), jnp.float32)],
        mesh=mesh,
        scratch_types=[
            pltpu.VMEM((CPT_DEG, K), jnp.int32),
            pltpu.VMEM((112,), jnp.float32),
            pltpu.VMEM((ROWS_T,), jnp.float32),
            pltpu.VMEM_SHARED((NP,), jnp.float32),
        ],
    )
    agg_kernel = pl.kernel(
        _agg_body,
        out_type=jax.ShapeDtypeStruct((2, NP, H), jnp.float32),
        mesh=mesh,
        scratch_types=[
            pltpu.VMEM((GCPT, K), jnp.int32),
            pltpu.VMEM((GCPT, K), jnp.int32),
            pltpu.VMEM((K, H), jnp.float32),
            pltpu.VMEM((K, H), jnp.float32),
            pltpu.VMEM((K, H), jnp.float32),
            pltpu.VMEM_SHARED((NP, H), jnp.float32),
            pltpu.SemaphoreType.DMA,
            pltpu.SemaphoreType.DMA,
            pltpu.SemaphoreType.DMA,
            pltpu.SemaphoreType.DMA,
            pltpu.SemaphoreType.DMA,
            pltpu.SemaphoreType.DMA,
        ],
    )
    return deg_kernel, agg_kernel


def _emb_body(x_ref, d0_ref, d1_ref, tcat_ref, w0_ref, p0_ref, dinv_ref):
    m0 = lax.dot_general(tcat_ref[...], w0_ref[...], (((1,), (0,)), ((), ())),
                         preferred_element_type=jnp.float32)
    xt = x_ref[...]
    iota = lax.broadcasted_iota(jnp.int32, (BLK, TPAD), 1)
    cmat = jnp.zeros((BLK, TPAD), jnp.float32)
    off = 0
    for i, d in enumerate(ATOM_DIMS):
        cmat = cmat + jnp.where(iota == xt[:, i:i + 1] + off, 1.0, 0.0)
        off += d
    h = lax.dot_general(cmat, m0, (((1,), (0,)), ((), ())),
                        preferred_element_type=jnp.float32)
    dinv = lax.rsqrt(d0_ref[...] + d1_ref[...] + 1.0).reshape(BLK, 1)
    p0_ref[...] = dinv * h
    dinv_ref[...] = dinv.reshape(BLK)


_emb_call = pl.pallas_call(
    _emb_body,
    grid=(NP // BLK,),
    in_specs=[
        pl.BlockSpec((BLK, 9), lambda i: (i, 0)),
        pl.BlockSpec((BLK,), lambda i: (i,)),
        pl.BlockSpec((BLK,), lambda i: (i,)),
        pl.BlockSpec((TPAD, H), lambda i: (0, 0)),
        pl.BlockSpec((H, H), lambda i: (0, 0)),
    ],
    out_specs=[
        pl.BlockSpec((BLK, H), lambda i: (i, 0)),
        pl.BlockSpec((BLK,), lambda i: (i,)),
    ],
    out_shape=[
        jax.ShapeDtypeStruct((NP, H), jnp.float32),
        jax.ShapeDtypeStruct((NP,), jnp.float32),
    ],
)


def _layer_body(acc_ref, p_ref, dinv_ref, b_ref, bng_ref, bnb_ref, w_ref,
                out_ref):
    dinv = dinv_ref[...].reshape(BLK, 1)
    agg = acc_ref[0] + acc_ref[1] + p_ref[...]
    conv = dinv * agg + b_ref[...]
    h = jnp.maximum(conv * (BN_SCALE * bng_ref[...]) + bnb_ref[...], 0.0)
    out_ref[...] = dinv * lax.dot_general(
        h, w_ref[...], (((1,), (0,)), ((), ())),
        preferred_element_type=jnp.float32)


_layer_call = pl.pallas_call(
    _layer_body,
    grid=(NP // BLK,),
    in_specs=[
        pl.BlockSpec((2, BLK, H), lambda i: (0, i, 0)),
        pl.BlockSpec((BLK, H), lambda i: (i, 0)),
        pl.BlockSpec((BLK,), lambda i: (i,)),
        pl.BlockSpec((1, H), lambda i: (0, 0)),
        pl.BlockSpec((1, H), lambda i: (0, 0)),
        pl.BlockSpec((1, H), lambda i: (0, 0)),
        pl.BlockSpec((H, H), lambda i: (0, 0)),
    ],
    out_specs=pl.BlockSpec((BLK, H), lambda i: (i, 0)),
    out_shape=jax.ShapeDtypeStruct((NP, H), jnp.float32),
)


def _final_body(acc_ref, p_ref, dinv_ref, b2_ref, batch_ref, linw_ref,
                linb_ref, out_ref, sums_ref, cnts_ref):
    i = pl.program_id(0)

    @pl.when(i == 0)
    def _():
        sums_ref[...] = jnp.zeros_like(sums_ref)
        cnts_ref[...] = jnp.zeros_like(cnts_ref)

    dinv = dinv_ref[...].reshape(BLK, 1)
    h = dinv * (acc_ref[0] + acc_ref[1] + p_ref[...]) + b2_ref[...]
    ind = jnp.where(
        batch_ref[...].reshape(BLK, 1)
        == lax.broadcasted_iota(jnp.int32, (BLK, G), 1), 1.0, 0.0)
    sums_ref[...] += lax.dot_general(ind, h, (((0,), (0,)), ((), ())),
                                     preferred_element_type=jnp.float32)
    cnts_ref[...] += lax.dot_general(ind, jnp.ones((BLK, 1), jnp.float32),
                                     (((0,), (0,)), ((), ())),
                                     preferred_element_type=jnp.float32)

    @pl.when(i == pl.num_programs(0) - 1)
    def _():
        feats = sums_ref[...] / jnp.maximum(cnts_ref[...], 1.0)
        z = lax.dot_general(feats, linw_ref[...], (((1,), (0,)), ((), ())),
                            preferred_element_type=jnp.float32) + linb_ref[...]
        out_ref[...] = 1.0 / (1.0 + jnp.exp(-z))


_final_call = pl.pallas_call(
    _final_body,
    grid=(NP // BLK,),
    in_specs=[
        pl.BlockSpec((2, BLK, H), lambda i: (0, i, 0)),
        pl.BlockSpec((BLK, H), lambda i: (i, 0)),
        pl.BlockSpec((BLK,), lambda i: (i,)),
        pl.BlockSpec((1, H), lambda i: (0, 0)),
        pl.BlockSpec((BLK,), lambda i: (i,)),
        pl.BlockSpec((H, 1), lambda i: (0, 0)),
        pl.BlockSpec((1, 1), lambda i: (0, 0)),
    ],
    out_specs=pl.BlockSpec((G, 1), lambda i: (0, 0)),
    out_shape=jax.ShapeDtypeStruct((G, 1), jnp.float32),
    scratch_shapes=[
        pltpu.VMEM((G, H), jnp.float32),
        pltpu.VMEM((G, 1), jnp.float32),
    ],
)


def kernel(x, edge_index, batch, emb0, emb1, emb2, emb3, emb4, emb5, emb6,
           emb7, emb8, W0, b0, W1, b1, W2, b2, bn0_g, bn0_b, bn1_g, bn1_b,
           lin_W, lin_b):
    deg_kernel, agg_kernel = _sc_kernels()
    dst3d_deg = edge_index[1].reshape(NT, CPT_DEG, K)
    deg0, deg1 = deg_kernel(dst3d_deg)
    src4d = edge_index[0].reshape(NT, NGRP, GCPT, K)
    dst4d = edge_index[1].reshape(NT, NGRP, GCPT, K)
    batch_p = jnp.pad(batch, (0, NP - N), constant_values=G)
    tcat = jnp.pad(
        jnp.concatenate([emb0, emb1, emb2, emb3, emb4, emb5, emb6, emb7,
                         emb8], axis=0), ((0, TPAD - TOT), (0, 0)))

    p0, dinv = _emb_call(x, deg0, deg1, tcat, W0)
    acc0 = agg_kernel(p0, src4d, dst4d)
    p1 = _layer_call(acc0, p0, dinv, b0.reshape(1, H), bn0_g.reshape(1, H),
                     bn0_b.reshape(1, H), W1)
    acc1 = agg_kernel(p1, src4d, dst4d)
    p2 = _layer_call(acc1, p1, dinv, b1.reshape(1, H), bn1_g.reshape(1, H),
                     bn1_b.reshape(1, H), W2)
    acc2 = agg_kernel(p2, src4d, dst4d)
    out = _final_call(acc2, p2, dinv, b2.reshape(1, H), batch_p, lin_W,
                      lin_b.reshape(1, 1))
    return out


# speedup vs baseline: 27.9310x; 1.0142x over previous
import functools

import numpy as np
import jax
import jax.numpy as jnp
from jax import lax
from jax.experimental import pallas as pl
from jax.experimental.pallas import tpu as pltpu
from jax.experimental.pallas import tpu_sc as plsc

N = 10000
NP = 10240
E = 320000
H = 128
G = 128
K = 100
NT = 32
CPT = E // NT // K
NGRP = 4
GCPT = CPT // NGRP
CPT_DEG = E // NT // K
ROWS_T = NP // 16
BLK = 2048
ATOM_DIMS = [119, 4, 12, 12, 10, 6, 6, 2, 2]
TOT = sum(ATOM_DIMS)
TPAD = 176
BN_SCALE = float(1.0 / np.sqrt(1.0 + 1e-5))

def _deg_body(dst3d, deg0_out, deg1_out, idx_v, ones_v, zero_v, acc_sh):
    c = lax.axis_index("c")
    s = lax.axis_index("s")
    wid = c * 16 + s

    for i in range(7):
        ones_v[pl.ds(i * 16, 16)] = jnp.ones((16,), jnp.float32)
    for i in range(ROWS_T // 16):
        zero_v[pl.ds(i * 16, 16)] = jnp.zeros((16,), jnp.float32)
    pltpu.sync_copy(zero_v, acc_sh.at[pl.ds(s * ROWS_T, ROWS_T)])
    plsc.subcore_barrier()
    pltpu.sync_copy(dst3d.at[wid], idx_v)

    def body(j, carry):
        pltpu.sync_copy(ones_v.at[pl.ds(0, K)], acc_sh.at[idx_v.at[j]],
                        add=True)
        return carry

    lax.fori_loop(0, CPT_DEG, body, 0)
    plsc.subcore_barrier()

    @pl.when(c == 0)
    def _():
        pltpu.sync_copy(acc_sh.at[pl.ds(s * ROWS_T, ROWS_T)],
                        deg0_out.at[pl.ds(s * ROWS_T, ROWS_T)])

    @pl.when(c == 1)
    def _():
        pltpu.sync_copy(acc_sh.at[pl.ds(s * ROWS_T, ROWS_T)],
                        deg1_out.at[pl.ds(s * ROWS_T, ROWS_T)])


def _agg_body(p_hbm, src4d, dst4d, out, sidx, didx, buf0, buf1, buf2,
              acc_sh, gsem0, gsem1, gsem2, ssem0, ssem1, ssem2):
    c = lax.axis_index("c")
    s = lax.axis_index("s")
    wid = c * 16 + s

    def zb(i, carry):
        buf0[i // 8, pl.ds((i % 8) * 16, 16)] = jnp.zeros((16,), jnp.float32)
        return carry

    lax.fori_loop(0, K * 8, zb, 0)
    for kk in range(ROWS_T // 80):
        pltpu.sync_copy(buf0.at[pl.ds(0, 80)],
                        acc_sh.at[pl.ds(s * ROWS_T + kk * 80, 80)])
    plsc.subcore_barrier()

    bufs = (buf0, buf1, buf2)
    gsems = (gsem0, gsem1, gsem2)
    ssems = (ssem0, ssem1, ssem2)
    for g in range(NGRP):
        pltpu.sync_copy(src4d.at[wid, g], sidx)
        pltpu.sync_copy(dst4d.at[wid, g], didx)
        pltpu.async_copy(p_hbm.at[sidx.at[0]], buf0, gsem0)
        pltpu.async_copy(p_hbm.at[sidx.at[1]], buf1, gsem1)

        def outer(jj, carry):
            for b in range(3):
                j = jj * 3 + b

                @pl.when(j < GCPT)
                def _():
                    pltpu.make_async_copy(p_hbm.at[sidx.at[j]], bufs[b],
                                          gsems[b]).wait()
                    pltpu.async_copy(bufs[b], acc_sh.at[didx.at[j]],
                                     ssems[b], add=True)
                    bn = (b + 2) % 3

                    @pl.when(j + 2 < GCPT)
                    def _():
                        @pl.when(j >= 1)
                        def _():
                            pltpu.make_async_copy(
                                bufs[bn], acc_sh.at[didx.at[j]],
                                ssems[bn]).wait()

                        pltpu.async_copy(p_hbm.at[sidx.at[j + 2]],
                                         bufs[bn], gsems[bn])
            return carry

        lax.fori_loop(0, (GCPT + 2) // 3, outer, 0)
        for b in range(3):
            pltpu.make_async_copy(bufs[b], acc_sh.at[didx.at[0]],
                                  ssems[b]).wait()
    plsc.subcore_barrier()
    pltpu.sync_copy(acc_sh.at[pl.ds(s * ROWS_T, ROWS_T)],
                    out.at[c, pl.ds(s * ROWS_T, ROWS_T)])


@functools.cache
def _sc_kernels():
    mesh = plsc.VectorSubcoreMesh(core_axis_name="c", subcore_axis_name="s",
                                  num_cores=2, num_subcores=16)
    deg_kernel = pl.kernel(
        _deg_body,
        out_type=[jax.ShapeDtypeStruct((NP,), jnp.float32),
                  jax.ShapeDtypeStruct((NP,), jnp.float32)],
        mesh=mesh,
        scratch_types=[
            pltpu.VMEM((CPT_DEG, K), jnp.int32),
            pltpu.VMEM((112,), jnp.float32),
            pltpu.VMEM((ROWS_T,), jnp.float32),
            pltpu.VMEM_SHARED((NP,), jnp.float32),
        ],
    )
    agg_kernel = pl.kernel(
        _agg_body,
        out_type=jax.ShapeDtypeStruct((2, NP, H), jnp.float32),
        mesh=mesh,
        scratch_types=[
            pltpu.VMEM((GCPT, K), jnp.int32),
            pltpu.VMEM((GCPT, K), jnp.int32),
            pltpu.VMEM((K, H), jnp.float32),
            pltpu.VMEM((K, H), jnp.float32),
            pltpu.VMEM((K, H), jnp.float32),
            pltpu.VMEM_SHARED((NP, H), jnp.float32),
            pltpu.SemaphoreType.DMA,
            pltpu.SemaphoreType.DMA,
            pltpu.SemaphoreType.DMA,
            pltpu.SemaphoreType.DMA,
            pltpu.SemaphoreType.DMA,
            pltpu.SemaphoreType.DMA,
        ],
    )
    return deg_kernel, agg_kernel


def _emb_body(x_ref, d0_ref, d1_ref, tcat_ref, w0_ref, p0_ref, dinv_ref):
    m0 = lax.dot_general(tcat_ref[...], w0_ref[...], (((1,), (0,)), ((), ())),
                         preferred_element_type=jnp.float32)
    xt = x_ref[...]
    iota = lax.broadcasted_iota(jnp.int32, (BLK, TPAD), 1)
    cmat = jnp.zeros((BLK, TPAD), jnp.float32)
    off = 0
    for i, d in enumerate(ATOM_DIMS):
        cmat = cmat + jnp.where(iota == xt[:, i:i + 1] + off, 1.0, 0.0)
        off += d
    h = lax.dot_general(cmat, m0, (((1,), (0,)), ((), ())),
                        preferred_element_type=jnp.float32)
    dinv = lax.rsqrt(d0_ref[...] + d1_ref[...] + 1.0).reshape(BLK, 1)
    p0_ref[...] = dinv * h
    dinv_ref[...] = dinv.reshape(BLK)


_emb_call = pl.pallas_call(
    _emb_body,
    grid=(NP // BLK,),
    in_specs=[
        pl.BlockSpec((BLK, 9), lambda i: (i, 0)),
        pl.BlockSpec((BLK,), lambda i: (i,)),
        pl.BlockSpec((BLK,), lambda i: (i,)),
        pl.BlockSpec((TPAD, H), lambda i: (0, 0)),
        pl.BlockSpec((H, H), lambda i: (0, 0)),
    ],
    out_specs=[
        pl.BlockSpec((BLK, H), lambda i: (i, 0)),
        pl.BlockSpec((BLK,), lambda i: (i,)),
    ],
    out_shape=[
        jax.ShapeDtypeStruct((NP, H), jnp.float32),
        jax.ShapeDtypeStruct((NP,), jnp.float32),
    ],
)


def _layer_body(acc_ref, p_ref, dinv_ref, b_ref, bng_ref, bnb_ref, w_ref,
                out_ref):
    dinv = dinv_ref[...].reshape(BLK, 1)
    agg = acc_ref[0] + acc_ref[1] + p_ref[...]
    conv = dinv * agg + b_ref[...]
    h = jnp.maximum(conv * (BN_SCALE * bng_ref[...]) + bnb_ref[...], 0.0)
    out_ref[...] = dinv * lax.dot_general(
        h, w_ref[...], (((1,), (0,)), ((), ())),
        preferred_element_type=jnp.float32)


_layer_call = pl.pallas_call(
    _layer_body,
    grid=(NP // BLK,),
    in_specs=[
        pl.BlockSpec((2, BLK, H), lambda i: (0, i, 0)),
        pl.BlockSpec((BLK, H), lambda i: (i, 0)),
        pl.BlockSpec((BLK,), lambda i: (i,)),
        pl.BlockSpec((1, H), lambda i: (0, 0)),
        pl.BlockSpec((1, H), lambda i: (0, 0)),
        pl.BlockSpec((1, H), lambda i: (0, 0)),
        pl.BlockSpec((H, H), lambda i: (0, 0)),
    ],
    out_specs=pl.BlockSpec((BLK, H), lambda i: (i, 0)),
    out_shape=jax.ShapeDtypeStruct((NP, H), jnp.float32),
)


def _final_body(acc_ref, p_ref, dinv_ref, b2_ref, batch_ref, linw_ref,
                linb_ref, out_ref, sums_ref, cnts_ref):
    i = pl.program_id(0)

    @pl.when(i == 0)
    def _():
        sums_ref[...] = jnp.zeros_like(sums_ref)
        cnts_ref[...] = jnp.zeros_like(cnts_ref)

    dinv = dinv_ref[...].reshape(BLK, 1)
    h = dinv * (acc_ref[0] + acc_ref[1] + p_ref[...]) + b2_ref[...]
    ind = jnp.where(
        batch_ref[...].reshape(BLK, 1)
        == lax.broadcasted_iota(jnp.int32, (BLK, G), 1), 1.0, 0.0)
    sums_ref[...] += lax.dot_general(ind, h, (((0,), (0,)), ((), ())),
                                     preferred_element_type=jnp.float32)
    cnts_ref[...] += lax.dot_general(ind, jnp.ones((BLK, 1), jnp.float32),
                                     (((0,), (0,)), ((), ())),
                                     preferred_element_type=jnp.float32)

    @pl.when(i == pl.num_programs(0) - 1)
    def _():
        feats = sums_ref[...] / jnp.maximum(cnts_ref[...], 1.0)
        z = lax.dot_general(feats, linw_ref[...], (((1,), (0,)), ((), ())),
                            preferred_element_type=jnp.float32) + linb_ref[...]
        out_ref[...] = 1.0 / (1.0 + jnp.exp(-z))


_final_call = pl.pallas_call(
    _final_body,
    grid=(NP // BLK,),
    in_specs=[
        pl.BlockSpec((2, BLK, H), lambda i: (0, i, 0)),
        pl.BlockSpec((BLK, H), lambda i: (i, 0)),
        pl.BlockSpec((BLK,), lambda i: (i,)),
        pl.BlockSpec((1, H), lambda i: (0, 0)),
        pl.BlockSpec((BLK,), lambda i: (i,)),
        pl.BlockSpec((H, 1), lambda i: (0, 0)),
        pl.BlockSpec((1, 1), lambda i: (0, 0)),
    ],
    out_specs=pl.BlockSpec((G, 1), lambda i: (0, 0)),
    out_shape=jax.ShapeDtypeStruct((G, 1), jnp.float32),
    scratch_shapes=[
        pltpu.VMEM((G, H), jnp.float32),
        pltpu.VMEM((G, 1), jnp.float32),
    ],
)


def kernel(x, edge_index, batch, emb0, emb1, emb2, emb3, emb4, emb5, emb6,
           emb7, emb8, W0, b0, W1, b1, W2, b2, bn0_g, bn0_b, bn1_g, bn1_b,
           lin_W, lin_b):
    deg_kernel, agg_kernel = _sc_kernels()
    dst3d_deg = edge_index[1].reshape(NT, CPT_DEG, K)
    deg0, deg1 = deg_kernel(dst3d_deg)
    src4d = edge_index[0].reshape(NT, NGRP, GCPT, K)
    dst4d = edge_index[1].reshape(NT, NGRP, GCPT, K)
    batch_p = jnp.pad(batch, (0, NP - N), constant_values=G)
    tcat = jnp.pad(
        jnp.concatenate([emb0, emb1, emb2, emb3, emb4, emb5, emb6, emb7,
                         emb8], axis=0), ((0, TPAD - TOT), (0, 0)))

    p0, dinv = _emb_call(x, deg0, deg1, tcat, W0)
    acc0 = agg_kernel(p0, src4d, dst4d)
    p1 = _layer_call(acc0, p0, dinv, b0.reshape(1, H), bn0_g.reshape(1, H),
                     bn0_b.reshape(1, H), W1)
    acc1 = agg_kernel(p1, src4d, dst4d)
    p2 = _layer_call(acc1, p1, dinv, b1.reshape(1, H), bn1_g.reshape(1, H),
                     bn1_b.reshape(1, H), W2)
    acc2 = agg_kernel(p2, src4d, dst4d)
    out = _final_call(acc2, p2, dinv, b2.reshape(1, H), batch_p, lin_W,
                      lin_b.reshape(1, 1))
    return out


# speedup vs baseline: 28.2546x; 1.0116x over previous
import functools

import numpy as np
import jax
import jax.numpy as jnp
from jax import lax
from jax.experimental import pallas as pl
from jax.experimental.pallas import tpu as pltpu
from jax.experimental.pallas import tpu_sc as plsc

N = 10000
NP = 10240
E = 320000
H = 128
G = 128
K = 100
NT = 32
CPT = E // NT // K
NGRP = 4
GCPT = CPT // NGRP
CPT_DEG = E // NT // K
ROWS_T = NP // 16
BLK = 5120
ATOM_DIMS = [119, 4, 12, 12, 10, 6, 6, 2, 2]
TOT = sum(ATOM_DIMS)
TPAD = 176
BN_SCALE = float(1.0 / np.sqrt(1.0 + 1e-5))

def _deg_body(dst3d, deg0_out, deg1_out, idx_v, ones_v, zero_v, acc_sh):
    c = lax.axis_index("c")
    s = lax.axis_index("s")
    wid = c * 16 + s

    for i in range(7):
        ones_v[pl.ds(i * 16, 16)] = jnp.ones((16,), jnp.float32)
    for i in range(ROWS_T // 16):
        zero_v[pl.ds(i * 16, 16)] = jnp.zeros((16,), jnp.float32)
    pltpu.sync_copy(zero_v, acc_sh.at[pl.ds(s * ROWS_T, ROWS_T)])
    plsc.subcore_barrier()
    pltpu.sync_copy(dst3d.at[wid], idx_v)

    def body(j, carry):
        pltpu.sync_copy(ones_v.at[pl.ds(0, K)], acc_sh.at[idx_v.at[j]],
                        add=True)
        return carry

    lax.fori_loop(0, CPT_DEG, body, 0)
    plsc.subcore_barrier()

    @pl.when(c == 0)
    def _():
        pltpu.sync_copy(acc_sh.at[pl.ds(s * ROWS_T, ROWS_T)],
                        deg0_out.at[pl.ds(s * ROWS_T, ROWS_T)])

    @pl.when(c == 1)
    def _():
        pltpu.sync_copy(acc_sh.at[pl.ds(s * ROWS_T, ROWS_T)],
                        deg1_out.at[pl.ds(s * ROWS_T, ROWS_T)])


def _agg_body(p_hbm, src4d, dst4d, out, sidx, didx, buf0, buf1, buf2,
              acc_sh, gsem0, gsem1, gsem2, ssem0, ssem1, ssem2):
    c = lax.axis_index("c")
    s = lax.axis_index("s")
    wid = c * 16 + s

    def zb(i, carry):
        buf0[i // 8, pl.ds((i % 8) * 16, 16)] = jnp.zeros((16,), jnp.float32)
        return carry

    lax.fori_loop(0, K * 8, zb, 0)
    for kk in range(ROWS_T // 80):
        pltpu.sync_copy(buf0.at[pl.ds(0, 80)],
                        acc_sh.at[pl.ds(s * ROWS_T + kk * 80, 80)])
    plsc.subcore_barrier()

    bufs = (buf0, buf1, buf2)
    gsems = (gsem0, gsem1, gsem2)
    ssems = (ssem0, ssem1, ssem2)
    for g in range(NGRP):
        pltpu.sync_copy(src4d.at[wid, g], sidx)
        pltpu.sync_copy(dst4d.at[wid, g], didx)
        pltpu.async_copy(p_hbm.at[sidx.at[0]], buf0, gsem0)
        pltpu.async_copy(p_hbm.at[sidx.at[1]], buf1, gsem1)

        def outer(jj, carry):
            for b in range(3):
                j = jj * 3 + b

                @pl.when(j < GCPT)
                def _():
                    pltpu.make_async_copy(p_hbm.at[sidx.at[j]], bufs[b],
                                          gsems[b]).wait()
                    pltpu.async_copy(bufs[b], acc_sh.at[didx.at[j]],
                                     ssems[b], add=True)
                    bn = (b + 2) % 3

                    @pl.when(j + 2 < GCPT)
                    def _():
                        @pl.when(j >= 1)
                        def _():
                            pltpu.make_async_copy(
                                bufs[bn], acc_sh.at[didx.at[j]],
                                ssems[bn]).wait()

                        pltpu.async_copy(p_hbm.at[sidx.at[j + 2]],
                                         bufs[bn], gsems[bn])
            return carry

        lax.fori_loop(0, (GCPT + 2) // 3, outer, 0)
        for b in range(3):
            pltpu.make_async_copy(bufs[b], acc_sh.at[didx.at[0]],
                                  ssems[b]).wait()
    plsc.subcore_barrier()
    pltpu.sync_copy(acc_sh.at[pl.ds(s * ROWS_T, ROWS_T)],
                    out.at[c, pl.ds(s * ROWS_T, ROWS_T)])


@functools.cache
def _sc_kernels():
    mesh = plsc.VectorSubcoreMesh(core_axis_name="c", subcore_axis_name="s",
                                  num_cores=2, num_subcores=16)
    deg_kernel = pl.kernel(
        _deg_body,
        out_type=[jax.ShapeDtypeStruct((NP,), jnp.float32),
                  jax.ShapeDtypeStruct((NP,), jnp.float32)],
        mesh=mesh,
        scratch_types=[
            pltpu.VMEM((CPT_DEG, K), jnp.int32),
            pltpu.VMEM((112,), jnp.float32),
            pltpu.VMEM((ROWS_T,), jnp.float32),
            pltpu.VMEM_SHARED((NP,), jnp.float32),
        ],
    )
    agg_kernel = pl.kernel(
        _agg_body,
        out_type=jax.ShapeDtypeStruct((2, NP, H), jnp.float32),
        mesh=mesh,
        scratch_types=[
            pltpu.VMEM((GCPT, K), jnp.int32),
            pltpu.VMEM((GCPT, K), jnp.int32),
            pltpu.VMEM((K, H), jnp.float32),
            pltpu.VMEM((K, H), jnp.float32),
            pltpu.VMEM((K, H), jnp.float32),
            pltpu.VMEM_SHARED((NP, H), jnp.float32),
            pltpu.SemaphoreType.DMA,
            pltpu.SemaphoreType.DMA,
            pltpu.SemaphoreType.DMA,
            pltpu.SemaphoreType.DMA,
            pltpu.SemaphoreType.DMA,
            pltpu.SemaphoreType.DMA,
        ],
    )
    return deg_kernel, agg_kernel


def _emb_body(x_ref, d0_ref, d1_ref, tcat_ref, w0_ref, p0_ref, dinv_ref):
    m0 = lax.dot_general(tcat_ref[...], w0_ref[...], (((1,), (0,)), ((), ())),
                         preferred_element_type=jnp.float32)
    xt = x_ref[...]
    iota = lax.broadcasted_iota(jnp.int32, (BLK, TPAD), 1)
    cmat = jnp.zeros((BLK, TPAD), jnp.float32)
    off = 0
    for i, d in enumerate(ATOM_DIMS):
        cmat = cmat + jnp.where(iota == xt[:, i:i + 1] + off, 1.0, 0.0)
        off += d
    h = lax.dot_general(cmat, m0, (((1,), (0,)), ((), ())),
                        preferred_element_type=jnp.float32)
    dinv = lax.rsqrt(d0_ref[...] + d1_ref[...] + 1.0).reshape(BLK, 1)
    p0_ref[...] = dinv * h
    dinv_ref[...] = dinv.reshape(BLK)


_emb_call = pl.pallas_call(
    _emb_body,
    grid=(NP // BLK,),
    in_specs=[
        pl.BlockSpec((BLK, 9), lambda i: (i, 0)),
        pl.BlockSpec((BLK,), lambda i: (i,)),
        pl.BlockSpec((BLK,), lambda i: (i,)),
        pl.BlockSpec((TPAD, H), lambda i: (0, 0)),
        pl.BlockSpec((H, H), lambda i: (0, 0)),
    ],
    out_specs=[
        pl.BlockSpec((BLK, H), lambda i: (i, 0)),
        pl.BlockSpec((BLK,), lambda i: (i,)),
    ],
    out_shape=[
        jax.ShapeDtypeStruct((NP, H), jnp.float32),
        jax.ShapeDtypeStruct((NP,), jnp.float32),
    ],
)


def _layer_body(acc_ref, p_ref, dinv_ref, b_ref, bng_ref, bnb_ref, w_ref,
                out_ref):
    dinv = dinv_ref[...].reshape(BLK, 1)
    agg = acc_ref[0] + acc_ref[1] + p_ref[...]
    conv = dinv * agg + b_ref[...]
    h = jnp.maximum(conv * (BN_SCALE * bng_ref[...]) + bnb_ref[...], 0.0)
    out_ref[...] = dinv * lax.dot_general(
        h, w_ref[...], (((1,), (0,)), ((), ())),
        preferred_element_type=jnp.float32)


_layer_call = pl.pallas_call(
    _layer_body,
    grid=(NP // BLK,),
    in_specs=[
        pl.BlockSpec((2, BLK, H), lambda i: (0, i, 0)),
        pl.BlockSpec((BLK, H), lambda i: (i, 0)),
        pl.BlockSpec((BLK,), lambda i: (i,)),
        pl.BlockSpec((1, H), lambda i: (0, 0)),
        pl.BlockSpec((1, H), lambda i: (0, 0)),
        pl.BlockSpec((1, H), lambda i: (0, 0)),
        pl.BlockSpec((H, H), lambda i: (0, 0)),
    ],
    out_specs=pl.BlockSpec((BLK, H), lambda i: (i, 0)),
    out_shape=jax.ShapeDtypeStruct((NP, H), jnp.float32),
)


def _final_body(acc_ref, p_ref, dinv_ref, b2_ref, batch_ref, linw_ref,
                linb_ref, out_ref, sums_ref, cnts_ref):
    i = pl.program_id(0)

    @pl.when(i == 0)
    def _():
        sums_ref[...] = jnp.zeros_like(sums_ref)
        cnts_ref[...] = jnp.zeros_like(cnts_ref)

    dinv = dinv_ref[...].reshape(BLK, 1)
    h = dinv * (acc_ref[0] + acc_ref[1] + p_ref[...]) + b2_ref[...]
    ind = jnp.where(
        batch_ref[...].reshape(BLK, 1)
        == lax.broadcasted_iota(jnp.int32, (BLK, G), 1), 1.0, 0.0)
    sums_ref[...] += lax.dot_general(ind, h, (((0,), (0,)), ((), ())),
                                     preferred_element_type=jnp.float32)
    cnts_ref[...] += lax.dot_general(ind, jnp.ones((BLK, 1), jnp.float32),
                                     (((0,), (0,)), ((), ())),
                                     preferred_element_type=jnp.float32)

    @pl.when(i == pl.num_programs(0) - 1)
    def _():
        feats = sums_ref[...] / jnp.maximum(cnts_ref[...], 1.0)
        z = lax.dot_general(feats, linw_ref[...], (((1,), (0,)), ((), ())),
                            preferred_element_type=jnp.float32) + linb_ref[...]
        out_ref[...] = 1.0 / (1.0 + jnp.exp(-z))


_final_call = pl.pallas_call(
    _final_body,
    grid=(NP // BLK,),
    in_specs=[
        pl.BlockSpec((2, BLK, H), lambda i: (0, i, 0)),
        pl.BlockSpec((BLK, H), lambda i: (i, 0)),
        pl.BlockSpec((BLK,), lambda i: (i,)),
        pl.BlockSpec((1, H), lambda i: (0, 0)),
        pl.BlockSpec((BLK,), lambda i: (i,)),
        pl.BlockSpec((H, 1), lambda i: (0, 0)),
        pl.BlockSpec((1, 1), lambda i: (0, 0)),
    ],
    out_specs=pl.BlockSpec((G, 1), lambda i: (0, 0)),
    out_shape=jax.ShapeDtypeStruct((G, 1), jnp.float32),
    scratch_shapes=[
        pltpu.VMEM((G, H), jnp.float32),
        pltpu.VMEM((G, 1), jnp.float32),
    ],
)


def kernel(x, edge_index, batch, emb0, emb1, emb2, emb3, emb4, emb5, emb6,
           emb7, emb8, W0, b0, W1, b1, W2, b2, bn0_g, bn0_b, bn1_g, bn1_b,
           lin_W, lin_b):
    deg_kernel, agg_kernel = _sc_kernels()
    dst3d_deg = edge_index[1].reshape(NT, CPT_DEG, K)
    deg0, deg1 = deg_kernel(dst3d_deg)
    src4d = edge_index[0].reshape(NT, NGRP, GCPT, K)
    dst4d = edge_index[1].reshape(NT, NGRP, GCPT, K)
    batch_p = jnp.pad(batch, (0, NP - N), constant_values=G)
    tcat = jnp.pad(
        jnp.concatenate([emb0, emb1, emb2, emb3, emb4, emb5, emb6, emb7,
                         emb8], axis=0), ((0, TPAD - TOT), (0, 0)))

    p0, dinv = _emb_call(x, deg0, deg1, tcat, W0)
    acc0 = agg_kernel(p0, src4d, dst4d)
    p1 = _layer_call(acc0, p0, dinv, b0.reshape(1, H), bn0_g.reshape(1, H),
                     bn0_b.reshape(1, H), W1)
    acc1 = agg_kernel(p1, src4d, dst4d)
    p2 = _layer_call(acc1, p1, dinv, b1.reshape(1, H), bn1_g.reshape(1, H),
                     bn1_b.reshape(1, H), W2)
    acc2 = agg_kernel(p2, src4d, dst4d)
    out = _final_call(acc2, p2, dinv, b2.reshape(1, H), batch_p, lin_W,
                      lin_b.reshape(1, 1))
    return out
